# Initial kernel scaffold; baseline (speedup 1.0000x reference)
import jax
import jax.numpy as jnp
from jax.experimental import pallas as pl


def kernel(x, lower_edge_index, upper_edge_index, W_low, a_src_low, a_dst_low, W_up, a_src_up, a_dst_up, W_skip):
    raise NotImplementedError("write your pallas kernel here")


# speedup vs baseline: 13.1575x; 13.1575x over previous
import jax
import jax.numpy as jnp
from jax import lax
from jax.experimental import pallas as pl
from jax.experimental.pallas import tpu as pltpu
from jax.experimental.pallas import tpu_sc as plsc

N = 10000
E = 320000
D = 128
NC = 2
NS = 16
LANES = 16
K = 80
EPT = E // NS
SBC = 25
SBE = SBC * K
NSB = EPT // SBE
RPT = N // NS
BN = 1024

_SC_PARAMS = pltpu.CompilerParams(use_tc_tiling_on_sc=False,
                                  needs_layout_passes=False)


def _pre_body(x_ref, wl_ref, wu_ref, ws_ref, av_ref,
              xw_ref, sk_ref, st_ref):
    xb = x_ref[...]
    xwl = jnp.dot(xb, wl_ref[...], preferred_element_type=jnp.float32)
    xwu = jnp.dot(xb, wu_ref[...], preferred_element_type=jnp.float32)
    sk = jnp.dot(xb, ws_ref[...], preferred_element_type=jnp.float32)
    xw_ref[0] = xwl
    xw_ref[1] = xwu
    sk_ref[...] = sk * (1.0 + 1e-6)
    av = av_ref[...]
    stl = lax.dot_general(av[0:2], xwl, (((1,), (1,)), ((), ())),
                          preferred_element_type=jnp.float32)
    stu = lax.dot_general(av[2:4], xwu, (((1,), (1,)), ((), ())),
                          preferred_element_type=jnp.float32)
    st_ref[0] = stl
    st_ref[1] = stu


_pre = pl.pallas_call(
    _pre_body,
    grid=((N + BN - 1) // BN,),
    in_specs=[
        pl.BlockSpec((BN, D), lambda i: (i, 0)),
        pl.BlockSpec((D, D), lambda i: (0, 0)),
        pl.BlockSpec((D, D), lambda i: (0, 0)),
        pl.BlockSpec((D, D), lambda i: (0, 0)),
        pl.BlockSpec((4, D), lambda i: (0, 0)),
    ],
    out_specs=[
        pl.BlockSpec((2, BN, D), lambda i: (0, i, 0)),
        pl.BlockSpec((BN, D), lambda i: (i, 0)),
        pl.BlockSpec((2, 2, BN), lambda i: (0, 0, i)),
    ],
    out_shape=[
        jax.ShapeDtypeStruct((2, N, D), jnp.float32),
        jax.ShapeDtypeStruct((N, D), jnp.float32),
        jax.ShapeDtypeStruct((2, 2, N), jnp.float32),
    ],
)


def _sc1_body(edges, st, w_out, den_out,
              tgt_v, src_v, s_v, t_v, den_v, w_v):
    c = lax.axis_index("c")
    s = lax.axis_index("s")
    pltpu.sync_copy(st.at[c, 0], s_v)
    pltpu.sync_copy(st.at[c, 1], t_v)

    zv = jnp.zeros((LANES,), jnp.float32)

    @pl.loop(0, N // LANES)
    def _zero_den(i):
        den_v[pl.ds(i * LANES, LANES)] = zv

    @pl.loop(0, NSB)
    def _superchunk(j):
        pltpu.sync_copy(edges.at[c, 0, s, j], tgt_v)
        pltpu.sync_copy(edges.at[c, 1, s, j], src_v)

        @pl.loop(0, SBE // LANES)
        def _group(g):
            sl = pl.ds(g * LANES, LANES)
            tg = tgt_v[sl]
            sr = src_v[sl]
            a = plsc.load_gather(s_v, [sr]) + plsc.load_gather(t_v, [tg])
            a = jnp.maximum(a, a * 0.01)
            a = jnp.exp(a)
            plsc.addupdate_scatter(den_v, [tg], a)
            w_v[sl] = a

        pltpu.sync_copy(w_v, w_out.at[c, s, j])

    pltpu.sync_copy(den_v, den_out.at[c, s])


_sc1 = pl.kernel(
    _sc1_body,
    out_type=[
        jax.ShapeDtypeStruct((2, NS, NSB, SBE), jnp.float32),
        jax.ShapeDtypeStruct((2, NS, N), jnp.float32),
    ],
    mesh=plsc.VectorSubcoreMesh(core_axis_name="c", subcore_axis_name="s",
                                num_cores=NC, num_subcores=NS),
    compiler_params=_SC_PARAMS,
    scratch_types=[
        pltpu.VMEM((SBE,), jnp.int32),
        pltpu.VMEM((SBE,), jnp.int32),
        pltpu.VMEM((N,), jnp.float32),
        pltpu.VMEM((N,), jnp.float32),
        pltpu.VMEM((N,), jnp.float32),
        pltpu.VMEM((SBE,), jnp.float32),
    ],
)


def _sc2_body(edges6, w_in, xw, agg_out,
              tgt_sb, src_sb, w_v, rows0, agg_s, gsem, ssem):
    c = lax.axis_index("c")
    s = lax.axis_index("s")

    zv = jnp.zeros((LANES,), jnp.float32)

    @pl.loop(0, K)
    def _zero_rows(i):
        for k in range(D // LANES):
            rows0[i, pl.ds(k * LANES, LANES)] = zv

    for j in range(RPT // K):
        pltpu.sync_copy(rows0, agg_s.at[pl.ds(s * RPT + j * K, K)])
    pltpu.sync_copy(rows0.at[pl.ds(0, RPT - (RPT // K) * K)],
                    agg_s.at[pl.ds(s * RPT + (RPT // K) * K,
                                   RPT - (RPT // K) * K)])
    plsc.subcore_barrier()

    xw_c = xw.at[c]

    @pl.loop(0, NSB)
    def _superchunk(j):
        pltpu.sync_copy(edges6.at[c, 0, s, j], tgt_sb)
        pltpu.sync_copy(edges6.at[c, 1, s, j], src_sb)
        pltpu.sync_copy(w_in.at[c, s, j], w_v)

        @pl.loop(0, SBC)
        def _chunk(cc):
            pltpu.async_copy(xw_c.at[src_sb.at[cc]], rows0, gsem).wait()
            for e in range(K):
                widx = jnp.full((LANES,), cc * K + e, jnp.int32)
                w = plsc.load_gather(w_v, [widx])
                for k in range(D // LANES):
                    sl = pl.ds(k * LANES, LANES)
                    rows0[e, sl] = rows0[e, sl] * w
            pltpu.async_copy(rows0, agg_s.at[tgt_sb.at[cc]], ssem,
                             add=True).wait()

    plsc.subcore_barrier()
    pltpu.sync_copy(agg_s.at[pl.ds(s * RPT, RPT)],
                    agg_out.at[c, pl.ds(s * RPT, RPT)])


_sc2 = pl.kernel(
    _sc2_body,
    out_type=jax.ShapeDtypeStruct((2, N, D), jnp.float32),
    mesh=plsc.VectorSubcoreMesh(core_axis_name="c", subcore_axis_name="s",
                                num_cores=NC, num_subcores=NS),
    compiler_params=_SC_PARAMS,
    scratch_types=[
        pltpu.VMEM((SBC, K), jnp.int32),
        pltpu.VMEM((SBC, K), jnp.int32),
        pltpu.VMEM((SBE,), jnp.float32),
        pltpu.VMEM((K, D), jnp.float32),
        pltpu.VMEM_SHARED((N, D), jnp.float32),
        pltpu.SemaphoreType.DMA,
        pltpu.SemaphoreType.DMA,
    ],
)


def _post_body(agg_ref, den_ref, sk_ref, out_ref):
    dl = jnp.sum(den_ref[0], axis=0)
    du = jnp.sum(den_ref[1], axis=0)
    rl = 1.0 / (dl + 1e-16)
    ru = 1.0 / (du + 1e-16)
    o = agg_ref[0] * rl[:, None] + agg_ref[1] * ru[:, None] + sk_ref[...]
    out_ref[...] = jnp.maximum(o, 0.0)


_post = pl.pallas_call(
    _post_body,
    grid=((N + BN - 1) // BN,),
    in_specs=[
        pl.BlockSpec((2, BN, D), lambda i: (0, i, 0)),
        pl.BlockSpec((2, NS, BN), lambda i: (0, 0, i)),
        pl.BlockSpec((BN, D), lambda i: (i, 0)),
    ],
    out_specs=pl.BlockSpec((BN, D), lambda i: (i, 0)),
    out_shape=jax.ShapeDtypeStruct((N, D), jnp.float32),
)


@jax.jit
def kernel(x, lower_edge_index, upper_edge_index,
           W_low, a_src_low, a_dst_low,
           W_up, a_src_up, a_dst_up,
           W_skip):
    av = jnp.concatenate([
        a_src_low.reshape(1, D), a_dst_low.reshape(1, D),
        a_src_up.reshape(1, D), a_dst_up.reshape(1, D)], axis=0)
    xw, sk, st = _pre(x, W_low, W_up, W_skip, av)
    edges = jnp.stack([lower_edge_index, upper_edge_index])
    edges5 = edges.reshape(2, 2, NS, NSB, SBE)
    edges6 = edges.reshape(2, 2, NS, NSB, SBC, K)
    w, den = _sc1(edges5, st)
    agg = _sc2(edges6, w, xw)
    return _post(agg, den, sk)


# speedup vs baseline: 13.3992x; 1.0184x over previous
import jax
import jax.numpy as jnp
from jax import lax
from jax.experimental import pallas as pl
from jax.experimental.pallas import tpu as pltpu
from jax.experimental.pallas import tpu_sc as plsc

N = 10000
E = 320000
D = 128
NC = 2
NS = 16
LANES = 16
K = 80
EPT = E // NS
SBC = 50
SBE = SBC * K
NSB = EPT // SBE
RPT = N // NS
BN = 1024

_SC_PARAMS = pltpu.CompilerParams(use_tc_tiling_on_sc=False,
                                  needs_layout_passes=False)


def _pre_body(x_ref, wl_ref, wu_ref, ws_ref, av_ref,
              xw_ref, sk_ref, st_ref):
    xb = x_ref[...]
    xwl = jnp.dot(xb, wl_ref[...], preferred_element_type=jnp.float32)
    xwu = jnp.dot(xb, wu_ref[...], preferred_element_type=jnp.float32)
    sk = jnp.dot(xb, ws_ref[...], preferred_element_type=jnp.float32)
    xw_ref[0] = xwl
    xw_ref[1] = xwu
    sk_ref[...] = sk * (1.0 + 1e-6)
    av = av_ref[...]
    stl = lax.dot_general(av[0:2], xwl, (((1,), (1,)), ((), ())),
                          preferred_element_type=jnp.float32)
    stu = lax.dot_general(av[2:4], xwu, (((1,), (1,)), ((), ())),
                          preferred_element_type=jnp.float32)
    st_ref[0] = stl
    st_ref[1] = stu


_pre = pl.pallas_call(
    _pre_body,
    grid=((N + BN - 1) // BN,),
    in_specs=[
        pl.BlockSpec((BN, D), lambda i: (i, 0)),
        pl.BlockSpec((D, D), lambda i: (0, 0)),
        pl.BlockSpec((D, D), lambda i: (0, 0)),
        pl.BlockSpec((D, D), lambda i: (0, 0)),
        pl.BlockSpec((4, D), lambda i: (0, 0)),
    ],
    out_specs=[
        pl.BlockSpec((2, BN, D), lambda i: (0, i, 0)),
        pl.BlockSpec((BN, D), lambda i: (i, 0)),
        pl.BlockSpec((2, 2, BN), lambda i: (0, 0, i)),
    ],
    out_shape=[
        jax.ShapeDtypeStruct((2, N, D), jnp.float32),
        jax.ShapeDtypeStruct((N, D), jnp.float32),
        jax.ShapeDtypeStruct((2, 2, N), jnp.float32),
    ],
)


def _sc1_body(edges, st, w_out, den_out,
              tgt_v, src_v, s_v, t_v, den_v, w_v):
    c = lax.axis_index("c")
    s = lax.axis_index("s")
    pltpu.sync_copy(st.at[c, 0], s_v)
    pltpu.sync_copy(st.at[c, 1], t_v)

    zv = jnp.zeros((LANES,), jnp.float32)

    @pl.loop(0, N // LANES)
    def _zero_den(i):
        den_v[pl.ds(i * LANES, LANES)] = zv

    @pl.loop(0, NSB)
    def _superchunk(j):
        pltpu.sync_copy(edges.at[c, 0, s, j], tgt_v)
        pltpu.sync_copy(edges.at[c, 1, s, j], src_v)

        @pl.loop(0, SBE // LANES)
        def _group(g):
            sl = pl.ds(g * LANES, LANES)
            tg = tgt_v[sl]
            sr = src_v[sl]
            a = plsc.load_gather(s_v, [sr]) + plsc.load_gather(t_v, [tg])
            a = jnp.maximum(a, a * 0.01)
            a = jnp.exp(a)
            plsc.addupdate_scatter(den_v, [tg], a)
            w_v[sl] = a

        pltpu.sync_copy(w_v, w_out.at[c, s, j])

    pltpu.sync_copy(den_v, den_out.at[c, s])


_sc1 = pl.kernel(
    _sc1_body,
    out_type=[
        jax.ShapeDtypeStruct((2, NS, NSB, SBE), jnp.float32),
        jax.ShapeDtypeStruct((2, NS, N), jnp.float32),
    ],
    mesh=plsc.VectorSubcoreMesh(core_axis_name="c", subcore_axis_name="s",
                                num_cores=NC, num_subcores=NS),
    compiler_params=_SC_PARAMS,
    scratch_types=[
        pltpu.VMEM((SBE,), jnp.int32),
        pltpu.VMEM((SBE,), jnp.int32),
        pltpu.VMEM((N,), jnp.float32),
        pltpu.VMEM((N,), jnp.float32),
        pltpu.VMEM((N,), jnp.float32),
        pltpu.VMEM((SBE,), jnp.float32),
    ],
)


def _sc2_body(edges6, w_in, xw, agg_out,
              tgt_sb, src_sb, w_v, rows0, rows1, agg_s,
              gsem0, gsem1, ssem0, ssem1):
    c = lax.axis_index("c")
    s = lax.axis_index("s")

    zv = jnp.zeros((LANES,), jnp.float32)

    @pl.loop(0, K)
    def _zero_rows(i):
        for k in range(D // LANES):
            rows0[i, pl.ds(k * LANES, LANES)] = zv

    for j in range(RPT // K):
        pltpu.sync_copy(rows0, agg_s.at[pl.ds(s * RPT + j * K, K)])
    pltpu.sync_copy(rows0.at[pl.ds(0, RPT - (RPT // K) * K)],
                    agg_s.at[pl.ds(s * RPT + (RPT // K) * K,
                                   RPT - (RPT // K) * K)])
    plsc.subcore_barrier()

    xw_c = xw.at[c]
    bufs = ((rows0, gsem0, ssem0), (rows1, gsem1, ssem1))

    def _issue_gather(cc, b):
        rows, gsem, _ = bufs[b]
        pltpu.async_copy(xw_c.at[src_sb.at[cc]], rows, gsem)

    def _wait_gather(b):
        rows, gsem, _ = bufs[b]
        pltpu.make_async_copy(xw_c.at[src_sb.at[0]], rows, gsem).wait()

    def _issue_scatter(cc, b):
        rows, _, ssem = bufs[b]
        pltpu.async_copy(rows, agg_s.at[tgt_sb.at[cc]], ssem, add=True)

    def _wait_scatter(b):
        rows, _, ssem = bufs[b]
        pltpu.make_async_copy(rows, agg_s.at[tgt_sb.at[0]], ssem).wait()

    def _scale(cc, b):
        rows = bufs[b][0]
        base = cc * K
        for e in range(K):
            widx = jnp.full((LANES,), base + e, jnp.int32)
            w = plsc.load_gather(w_v, [widx])
            for k in range(D // LANES):
                sl = pl.ds(k * LANES, LANES)
                rows[e, sl] = rows[e, sl] * w

    @pl.loop(0, NSB)
    def _superchunk(j):
        pltpu.sync_copy(edges6.at[c, 0, s, j], tgt_sb)
        pltpu.sync_copy(edges6.at[c, 1, s, j], src_sb)
        pltpu.sync_copy(w_in.at[c, s, j], w_v)

        _issue_gather(0, 0)

        @pl.loop(0, SBC // 2)
        def _pair(p):
            cc0 = p * 2
            for b in range(2):
                cc = cc0 + b
                nb = 1 - b
                @pl.when(cc + 1 < SBC)
                def _():
                    @pl.when(cc >= 1)
                    def _():
                        _wait_scatter(nb)
                    _issue_gather(cc + 1, nb)
                _wait_gather(b)
                _scale(cc, b)
                _issue_scatter(cc, b)

        _wait_scatter(0)
        _wait_scatter(1)

    plsc.subcore_barrier()
    pltpu.sync_copy(agg_s.at[pl.ds(s * RPT, RPT)],
                    agg_out.at[c, pl.ds(s * RPT, RPT)])


_sc2 = pl.kernel(
    _sc2_body,
    out_type=jax.ShapeDtypeStruct((2, N, D), jnp.float32),
    mesh=plsc.VectorSubcoreMesh(core_axis_name="c", subcore_axis_name="s",
                                num_cores=NC, num_subcores=NS),
    compiler_params=_SC_PARAMS,
    scratch_types=[
        pltpu.VMEM((SBC, K), jnp.int32),
        pltpu.VMEM((SBC, K), jnp.int32),
        pltpu.VMEM((SBE,), jnp.float32),
        pltpu.VMEM((K, D), jnp.float32),
        pltpu.VMEM((K, D), jnp.float32),
        pltpu.VMEM_SHARED((N, D), jnp.float32),
        pltpu.SemaphoreType.DMA,
        pltpu.SemaphoreType.DMA,
        pltpu.SemaphoreType.DMA,
        pltpu.SemaphoreType.DMA,
    ],
)


def _post_body(agg_ref, den_ref, sk_ref, out_ref):
    dl = jnp.sum(den_ref[0], axis=0)
    du = jnp.sum(den_ref[1], axis=0)
    rl = 1.0 / (dl + 1e-16)
    ru = 1.0 / (du + 1e-16)
    o = agg_ref[0] * rl[:, None] + agg_ref[1] * ru[:, None] + sk_ref[...]
    out_ref[...] = jnp.maximum(o, 0.0)


_post = pl.pallas_call(
    _post_body,
    grid=((N + BN - 1) // BN,),
    in_specs=[
        pl.BlockSpec((2, BN, D), lambda i: (0, i, 0)),
        pl.BlockSpec((2, NS, BN), lambda i: (0, 0, i)),
        pl.BlockSpec((BN, D), lambda i: (i, 0)),
    ],
    out_specs=pl.BlockSpec((BN, D), lambda i: (i, 0)),
    out_shape=jax.ShapeDtypeStruct((N, D), jnp.float32),
)


@jax.jit
def kernel(x, lower_edge_index, upper_edge_index,
           W_low, a_src_low, a_dst_low,
           W_up, a_src_up, a_dst_up,
           W_skip):
    av = jnp.concatenate([
        a_src_low.reshape(1, D), a_dst_low.reshape(1, D),
        a_src_up.reshape(1, D), a_dst_up.reshape(1, D)], axis=0)
    xw, sk, st = _pre(x, W_low, W_up, W_skip, av)
    edges = jnp.stack([lower_edge_index, upper_edge_index])
    edges5 = edges.reshape(2, 2, NS, NSB, SBE)
    edges6 = edges.reshape(2, 2, NS, NSB, SBC, K)
    w, den = _sc1(edges5, st)
    agg = _sc2(edges6, w, xw)
    return _post(agg, den, sk)


# speedup vs baseline: 23.7705x; 1.7740x over previous
import jax
import jax.numpy as jnp
from jax import lax
from jax.experimental import pallas as pl
from jax.experimental.pallas import tpu as pltpu
from jax.experimental.pallas import tpu_sc as plsc

N = 10000
E = 320000
D = 128
NC = 2
NS = 16
LANES = 16
K = 80
EPT = E // NS
SBC = 50
SBE = SBC * K
NSB = EPT // SBE
RPT = N // NS
BN = 1024

_SC_PARAMS = pltpu.CompilerParams(use_tc_tiling_on_sc=False,
                                  needs_layout_passes=False)


def _pre_body(x_ref, wl_ref, wu_ref, ws_ref, av_ref,
              xw_ref, sk_ref, st_ref):
    xb = x_ref[...]
    xwl = jnp.dot(xb, wl_ref[...], preferred_element_type=jnp.float32)
    xwu = jnp.dot(xb, wu_ref[...], preferred_element_type=jnp.float32)
    sk = jnp.dot(xb, ws_ref[...], preferred_element_type=jnp.float32)
    xw_ref[0] = xwl
    xw_ref[1] = xwu
    sk_ref[...] = sk * (1.0 + 1e-6)
    av = av_ref[...]
    stl = lax.dot_general(av[0:2], xwl, (((1,), (1,)), ((), ())),
                          preferred_element_type=jnp.float32)
    stu = lax.dot_general(av[2:4], xwu, (((1,), (1,)), ((), ())),
                          preferred_element_type=jnp.float32)
    st_ref[0] = stl
    st_ref[1] = stu


_pre = pl.pallas_call(
    _pre_body,
    grid=((N + BN - 1) // BN,),
    in_specs=[
        pl.BlockSpec((BN, D), lambda i: (i, 0)),
        pl.BlockSpec((D, D), lambda i: (0, 0)),
        pl.BlockSpec((D, D), lambda i: (0, 0)),
        pl.BlockSpec((D, D), lambda i: (0, 0)),
        pl.BlockSpec((4, D), lambda i: (0, 0)),
    ],
    out_specs=[
        pl.BlockSpec((2, BN, D), lambda i: (0, i, 0)),
        pl.BlockSpec((BN, D), lambda i: (i, 0)),
        pl.BlockSpec((2, 2, BN), lambda i: (0, 0, i)),
    ],
    out_shape=[
        jax.ShapeDtypeStruct((2, N, D), jnp.float32),
        jax.ShapeDtypeStruct((N, D), jnp.float32),
        jax.ShapeDtypeStruct((2, 2, N), jnp.float32),
    ],
)


def _sc1_body(edges, st, w_out, den_out,
              tgt_v, src_v, s_v, t_v, den_v, w_v):
    c = lax.axis_index("c")
    s = lax.axis_index("s")
    pltpu.sync_copy(st.at[c, 0], s_v)
    pltpu.sync_copy(st.at[c, 1], t_v)

    zv = jnp.zeros((LANES,), jnp.float32)

    @pl.loop(0, N // LANES)
    def _zero_den(i):
        den_v[pl.ds(i * LANES, LANES)] = zv

    @pl.loop(0, NSB)
    def _superchunk(j):
        pltpu.sync_copy(edges.at[c, 0, s, j], tgt_v)
        pltpu.sync_copy(edges.at[c, 1, s, j], src_v)

        @pl.loop(0, SBE // LANES)
        def _group(g):
            sl = pl.ds(g * LANES, LANES)
            tg = tgt_v[sl]
            sr = src_v[sl]
            a = plsc.load_gather(s_v, [sr]) + plsc.load_gather(t_v, [tg])
            a = jnp.maximum(a, a * 0.01)
            a = jnp.exp(a)
            plsc.addupdate_scatter(den_v, [tg], a)
            w_v[sl] = a

        pltpu.sync_copy(w_v, w_out.at[c, s, j])

    pltpu.sync_copy(den_v, den_out.at[c, s])


_sc1 = pl.kernel(
    _sc1_body,
    out_type=[
        jax.ShapeDtypeStruct((2, NS, NSB, SBE), jnp.float32),
        jax.ShapeDtypeStruct((2, NS, N), jnp.float32),
    ],
    mesh=plsc.VectorSubcoreMesh(core_axis_name="c", subcore_axis_name="s",
                                num_cores=NC, num_subcores=NS),
    compiler_params=_SC_PARAMS,
    scratch_types=[
        pltpu.VMEM((SBE,), jnp.int32),
        pltpu.VMEM((SBE,), jnp.int32),
        pltpu.VMEM((N,), jnp.float32),
        pltpu.VMEM((N,), jnp.float32),
        pltpu.VMEM((N,), jnp.float32),
        pltpu.VMEM((SBE,), jnp.float32),
    ],
)


def _sc2_body(edges6, w_in, xw, agg_out,
              tgt_sb, src_sb, w_v, rows0, rows1, agg_s,
              gsem0, gsem1, ssem0, ssem1):
    c = lax.axis_index("c")
    s = lax.axis_index("s")

    zv = jnp.zeros((LANES,), jnp.float32)

    @pl.loop(0, K)
    def _zero_rows(i):
        for k in range(D // LANES):
            rows0[i, pl.ds(k * LANES, LANES)] = zv

    for j in range(RPT // K):
        pltpu.sync_copy(rows0, agg_s.at[pl.ds(s * RPT + j * K, K)])
    pltpu.sync_copy(rows0.at[pl.ds(0, RPT - (RPT // K) * K)],
                    agg_s.at[pl.ds(s * RPT + (RPT // K) * K,
                                   RPT - (RPT // K) * K)])
    plsc.subcore_barrier()

    xw_c = xw.at[c]
    bufs = ((rows0, gsem0, ssem0), (rows1, gsem1, ssem1))

    def _issue_gather(cc, b):
        rows, gsem, _ = bufs[b]
        pltpu.async_copy(xw_c.at[src_sb.at[cc]], rows, gsem)

    def _wait_gather(b):
        rows, gsem, _ = bufs[b]
        pltpu.make_async_copy(xw_c.at[src_sb.at[0]], rows, gsem).wait()

    def _issue_scatter(cc, b):
        rows, _, ssem = bufs[b]
        pltpu.async_copy(rows, agg_s.at[tgt_sb.at[cc]], ssem, add=True)

    def _wait_scatter(b):
        rows, _, ssem = bufs[b]
        pltpu.make_async_copy(rows, agg_s.at[tgt_sb.at[0]], ssem).wait()

    def _scale(cc, b):
        rows = bufs[b][0]
        base = cc * K

        @plsc.parallel_loop(0, K, unroll=8)
        def _edge(e):
            widx = jnp.full((LANES,), base + e, jnp.int32)
            w = plsc.load_gather(w_v, [widx])
            for k in range(D // LANES):
                sl = pl.ds(k * LANES, LANES)
                rows[e, sl] = rows[e, sl] * w

    @pl.loop(0, NSB)
    def _superchunk(j):
        pltpu.sync_copy(edges6.at[c, 0, s, j], tgt_sb)
        pltpu.sync_copy(edges6.at[c, 1, s, j], src_sb)
        pltpu.sync_copy(w_in.at[c, s, j], w_v)

        _issue_gather(0, 0)

        @pl.loop(0, SBC // 2)
        def _pair(p):
            cc0 = p * 2
            for b in range(2):
                cc = cc0 + b
                nb = 1 - b
                @pl.when(cc + 1 < SBC)
                def _():
                    @pl.when(cc >= 1)
                    def _():
                        _wait_scatter(nb)
                    _issue_gather(cc + 1, nb)
                _wait_gather(b)
                _scale(cc, b)
                _issue_scatter(cc, b)

        _wait_scatter(0)
        _wait_scatter(1)

    plsc.subcore_barrier()
    pltpu.sync_copy(agg_s.at[pl.ds(s * RPT, RPT)],
                    agg_out.at[c, pl.ds(s * RPT, RPT)])


_sc2 = pl.kernel(
    _sc2_body,
    out_type=jax.ShapeDtypeStruct((2, N, D), jnp.float32),
    mesh=plsc.VectorSubcoreMesh(core_axis_name="c", subcore_axis_name="s",
                                num_cores=NC, num_subcores=NS),
    compiler_params=_SC_PARAMS,
    scratch_types=[
        pltpu.VMEM((SBC, K), jnp.int32),
        pltpu.VMEM((SBC, K), jnp.int32),
        pltpu.VMEM((SBE,), jnp.float32),
        pltpu.VMEM((K, D), jnp.float32),
        pltpu.VMEM((K, D), jnp.float32),
        pltpu.VMEM_SHARED((N, D), jnp.float32),
        pltpu.SemaphoreType.DMA,
        pltpu.SemaphoreType.DMA,
        pltpu.SemaphoreType.DMA,
        pltpu.SemaphoreType.DMA,
    ],
)


def _post_body(agg_ref, den_ref, sk_ref, out_ref):
    dl = jnp.sum(den_ref[0], axis=0)
    du = jnp.sum(den_ref[1], axis=0)
    rl = 1.0 / (dl + 1e-16)
    ru = 1.0 / (du + 1e-16)
    o = agg_ref[0] * rl[:, None] + agg_ref[1] * ru[:, None] + sk_ref[...]
    out_ref[...] = jnp.maximum(o, 0.0)


_post = pl.pallas_call(
    _post_body,
    grid=((N + BN - 1) // BN,),
    in_specs=[
        pl.BlockSpec((2, BN, D), lambda i: (0, i, 0)),
        pl.BlockSpec((2, NS, BN), lambda i: (0, 0, i)),
        pl.BlockSpec((BN, D), lambda i: (i, 0)),
    ],
    out_specs=pl.BlockSpec((BN, D), lambda i: (i, 0)),
    out_shape=jax.ShapeDtypeStruct((N, D), jnp.float32),
)


@jax.jit
def kernel(x, lower_edge_index, upper_edge_index,
           W_low, a_src_low, a_dst_low,
           W_up, a_src_up, a_dst_up,
           W_skip):
    av = jnp.concatenate([
        a_src_low.reshape(1, D), a_dst_low.reshape(1, D),
        a_src_up.reshape(1, D), a_dst_up.reshape(1, D)], axis=0)
    xw, sk, st = _pre(x, W_low, W_up, W_skip, av)
    edges = jnp.stack([lower_edge_index, upper_edge_index])
    edges5 = edges.reshape(2, 2, NS, NSB, SBE)
    edges6 = edges.reshape(2, 2, NS, NSB, SBC, K)
    w, den = _sc1(edges5, st)
    agg = _sc2(edges6, w, xw)
    return _post(agg, den, sk)


# speedup vs baseline: 25.6940x; 1.0809x over previous
import jax
import jax.numpy as jnp
from jax import lax
from jax.experimental import pallas as pl
from jax.experimental.pallas import tpu as pltpu
from jax.experimental.pallas import tpu_sc as plsc

N = 10000
E = 320000
D = 128
NC = 2
NS = 16
LANES = 16
K = 80
EPT = E // NS
SBC = 25
SBE = SBC * K
NSB = EPT // SBE
NBUF = 4
RPT = N // NS
BN = 1024

_SC_PARAMS = pltpu.CompilerParams(use_tc_tiling_on_sc=False,
                                  needs_layout_passes=False)


def _pre_body(x_ref, wl_ref, wu_ref, ws_ref, av_ref,
              xw_ref, sk_ref, st_ref):
    xb = x_ref[...]
    xwl = jnp.dot(xb, wl_ref[...], preferred_element_type=jnp.float32)
    xwu = jnp.dot(xb, wu_ref[...], preferred_element_type=jnp.float32)
    sk = jnp.dot(xb, ws_ref[...], preferred_element_type=jnp.float32)
    xw_ref[0] = xwl
    xw_ref[1] = xwu
    sk_ref[...] = sk * (1.0 + 1e-6)
    av = av_ref[...]
    stl = lax.dot_general(av[0:2], xwl, (((1,), (1,)), ((), ())),
                          preferred_element_type=jnp.float32)
    stu = lax.dot_general(av[2:4], xwu, (((1,), (1,)), ((), ())),
                          preferred_element_type=jnp.float32)
    st_ref[0] = stl
    st_ref[1] = stu


_pre = pl.pallas_call(
    _pre_body,
    grid=((N + BN - 1) // BN,),
    in_specs=[
        pl.BlockSpec((BN, D), lambda i: (i, 0)),
        pl.BlockSpec((D, D), lambda i: (0, 0)),
        pl.BlockSpec((D, D), lambda i: (0, 0)),
        pl.BlockSpec((D, D), lambda i: (0, 0)),
        pl.BlockSpec((4, D), lambda i: (0, 0)),
    ],
    out_specs=[
        pl.BlockSpec((2, BN, D), lambda i: (0, i, 0)),
        pl.BlockSpec((BN, D), lambda i: (i, 0)),
        pl.BlockSpec((2, 2, BN), lambda i: (0, 0, i)),
    ],
    out_shape=[
        jax.ShapeDtypeStruct((2, N, D), jnp.float32),
        jax.ShapeDtypeStruct((N, D), jnp.float32),
        jax.ShapeDtypeStruct((2, 2, N), jnp.float32),
    ],
)


def _sc1_body(edges, st, w_out, den_out,
              tgt_v, src_v, s_v, t_v, den_v, w_v):
    c = lax.axis_index("c")
    s = lax.axis_index("s")
    pltpu.sync_copy(st.at[c, 0], s_v)
    pltpu.sync_copy(st.at[c, 1], t_v)

    zv = jnp.zeros((LANES,), jnp.float32)

    @pl.loop(0, N // LANES)
    def _zero_den(i):
        den_v[pl.ds(i * LANES, LANES)] = zv

    @pl.loop(0, NSB)
    def _superchunk(j):
        pltpu.sync_copy(edges.at[c, 0, s, j], tgt_v)
        pltpu.sync_copy(edges.at[c, 1, s, j], src_v)

        @plsc.parallel_loop(0, SBE // LANES, unroll=4)
        def _group(g):
            sl = pl.ds(g * LANES, LANES)
            tg = tgt_v[sl]
            sr = src_v[sl]
            a = plsc.load_gather(s_v, [sr]) + plsc.load_gather(t_v, [tg])
            a = jnp.maximum(a, a * 0.01)
            a = jnp.exp(a)
            plsc.addupdate_scatter(den_v, [tg], a)
            w_v[sl] = a

        pltpu.sync_copy(w_v, w_out.at[c, s, j])

    pltpu.sync_copy(den_v, den_out.at[c, s])


_sc1 = pl.kernel(
    _sc1_body,
    out_type=[
        jax.ShapeDtypeStruct((2, NS, NSB, SBE), jnp.float32),
        jax.ShapeDtypeStruct((2, NS, N), jnp.float32),
    ],
    mesh=plsc.VectorSubcoreMesh(core_axis_name="c", subcore_axis_name="s",
                                num_cores=NC, num_subcores=NS),
    compiler_params=_SC_PARAMS,
    scratch_types=[
        pltpu.VMEM((SBE,), jnp.int32),
        pltpu.VMEM((SBE,), jnp.int32),
        pltpu.VMEM((N,), jnp.float32),
        pltpu.VMEM((N,), jnp.float32),
        pltpu.VMEM((N,), jnp.float32),
        pltpu.VMEM((SBE,), jnp.float32),
    ],
)


def _sc2_body(edges6, w_in, xw, agg_out,
              tgt_sb, src_sb, w_v, rows0, rows1, rows2, rows3, agg_s,
              gsem0, gsem1, gsem2, gsem3, ssem0, ssem1, ssem2, ssem3):
    c = lax.axis_index("c")
    s = lax.axis_index("s")

    zv = jnp.zeros((LANES,), jnp.float32)

    @pl.loop(0, K)
    def _zero_rows(i):
        for k in range(D // LANES):
            rows0[i, pl.ds(k * LANES, LANES)] = zv

    for j in range(RPT // K):
        pltpu.sync_copy(rows0, agg_s.at[pl.ds(s * RPT + j * K, K)])
    pltpu.sync_copy(rows0.at[pl.ds(0, RPT - (RPT // K) * K)],
                    agg_s.at[pl.ds(s * RPT + (RPT // K) * K,
                                   RPT - (RPT // K) * K)])
    plsc.subcore_barrier()

    xw_c = xw.at[c]
    bufs = ((rows0, gsem0, ssem0), (rows1, gsem1, ssem1),
            (rows2, gsem2, ssem2), (rows3, gsem3, ssem3))

    def _issue_gather(cc, b):
        rows, gsem, _ = bufs[b]
        pltpu.async_copy(xw_c.at[src_sb.at[cc]], rows, gsem)

    def _wait_gather(b):
        rows, gsem, _ = bufs[b]
        pltpu.make_async_copy(xw_c.at[src_sb.at[0]], rows, gsem).wait()

    def _issue_scatter(cc, b):
        rows, _, ssem = bufs[b]
        pltpu.async_copy(rows, agg_s.at[tgt_sb.at[cc]], ssem, add=True)

    def _wait_scatter(b):
        rows, _, ssem = bufs[b]
        pltpu.make_async_copy(rows, agg_s.at[tgt_sb.at[0]], ssem).wait()

    def _scale(cc, b):
        rows = bufs[b][0]
        base = cc * K

        @plsc.parallel_loop(0, K, unroll=8)
        def _edge(e):
            widx = jnp.full((LANES,), base + e, jnp.int32)
            w = plsc.load_gather(w_v, [widx])
            for k in range(D // LANES):
                sl = pl.ds(k * LANES, LANES)
                rows[e, sl] = rows[e, sl] * w

    @pl.loop(0, NSB)
    def _superchunk(j):
        pltpu.sync_copy(edges6.at[c, 0, s, j], tgt_sb)
        pltpu.sync_copy(edges6.at[c, 1, s, j], src_sb)
        pltpu.sync_copy(w_in.at[c, s, j], w_v)

        _issue_gather(0, 0)

        def _step(cc, b):
            nb = (b + 1) % NBUF
            @pl.when(cc + 1 < SBC)
            def _():
                @pl.when(cc + 1 >= NBUF)
                def _():
                    _wait_scatter(nb)
                _issue_gather(cc + 1, nb)
            _wait_gather(b)
            _scale(cc, b)
            _issue_scatter(cc, b)

        @pl.loop(0, SBC // NBUF)
        def _quad(p):
            for b in range(NBUF):
                _step(p * NBUF + b, b)
        for cc in range((SBC // NBUF) * NBUF, SBC):
            _step(cc, cc % NBUF)

        for b in range(NBUF):
            _wait_scatter(b)

    plsc.subcore_barrier()
    pltpu.sync_copy(agg_s.at[pl.ds(s * RPT, RPT)],
                    agg_out.at[c, pl.ds(s * RPT, RPT)])


_sc2 = pl.kernel(
    _sc2_body,
    out_type=jax.ShapeDtypeStruct((2, N, D), jnp.float32),
    mesh=plsc.VectorSubcoreMesh(core_axis_name="c", subcore_axis_name="s",
                                num_cores=NC, num_subcores=NS),
    compiler_params=_SC_PARAMS,
    scratch_types=[
        pltpu.VMEM((SBC, K), jnp.int32),
        pltpu.VMEM((SBC, K), jnp.int32),
        pltpu.VMEM((SBE,), jnp.float32),
        pltpu.VMEM((K, D), jnp.float32),
        pltpu.VMEM((K, D), jnp.float32),
        pltpu.VMEM((K, D), jnp.float32),
        pltpu.VMEM((K, D), jnp.float32),
        pltpu.VMEM_SHARED((N, D), jnp.float32),
        pltpu.SemaphoreType.DMA,
        pltpu.SemaphoreType.DMA,
        pltpu.SemaphoreType.DMA,
        pltpu.SemaphoreType.DMA,
        pltpu.SemaphoreType.DMA,
        pltpu.SemaphoreType.DMA,
        pltpu.SemaphoreType.DMA,
        pltpu.SemaphoreType.DMA,
    ],
)


def _post_body(agg_ref, den_ref, sk_ref, out_ref):
    dl = jnp.sum(den_ref[0], axis=0)
    du = jnp.sum(den_ref[1], axis=0)
    rl = 1.0 / (dl + 1e-16)
    ru = 1.0 / (du + 1e-16)
    o = agg_ref[0] * rl[:, None] + agg_ref[1] * ru[:, None] + sk_ref[...]
    out_ref[...] = jnp.maximum(o, 0.0)


_post = pl.pallas_call(
    _post_body,
    grid=((N + BN - 1) // BN,),
    in_specs=[
        pl.BlockSpec((2, BN, D), lambda i: (0, i, 0)),
        pl.BlockSpec((2, NS, BN), lambda i: (0, 0, i)),
        pl.BlockSpec((BN, D), lambda i: (i, 0)),
    ],
    out_specs=pl.BlockSpec((BN, D), lambda i: (i, 0)),
    out_shape=jax.ShapeDtypeStruct((N, D), jnp.float32),
)


@jax.jit
def kernel(x, lower_edge_index, upper_edge_index,
           W_low, a_src_low, a_dst_low,
           W_up, a_src_up, a_dst_up,
           W_skip):
    av = jnp.concatenate([
        a_src_low.reshape(1, D), a_dst_low.reshape(1, D),
        a_src_up.reshape(1, D), a_dst_up.reshape(1, D)], axis=0)
    xw, sk, st = _pre(x, W_low, W_up, W_skip, av)
    edges = jnp.stack([lower_edge_index, upper_edge_index])
    edges5 = edges.reshape(2, 2, NS, NSB, SBE)
    edges6 = edges.reshape(2, 2, NS, NSB, SBC, K)
    w, den = _sc1(edges5, st)
    agg = _sc2(edges6, w, xw)
    return _post(agg, den, sk)


# speedup vs baseline: 27.9092x; 1.0862x over previous
import jax
import jax.numpy as jnp
import numpy as np
from jax import lax
from jax.experimental import pallas as pl
from jax.experimental.pallas import tpu as pltpu
from jax.experimental.pallas import tpu_sc as plsc

N = 10000
E = 320000
D = 128
NC = 2
NS = 16
LANES = 16
K = 80
EPT = E // NS
SBC = 25
SBE = SBC * K
NSB = EPT // SBE
NIN = 4
NOUT = 2
GLEAD = 3
RPT = N // NS
BN = 1024

_SC_PARAMS = pltpu.CompilerParams(use_tc_tiling_on_sc=False,
                                  needs_layout_passes=False)

_PERM = np.empty(D, np.int32)
for _k in range(D // 32):
    for _i in range(16):
        _PERM[32 * _k + 2 * _i] = 32 * _k + _i
        _PERM[32 * _k + 2 * _i + 1] = 32 * _k + 16 + _i


def _pre_body(x_ref, wl_ref, wu_ref, ws_ref, av_ref,
              xw_ref, sk_ref, st_ref):
    xb = x_ref[...]
    xwl = jnp.dot(xb, wl_ref[...], preferred_element_type=jnp.float32)
    xwu = jnp.dot(xb, wu_ref[...], preferred_element_type=jnp.float32)
    sk = jnp.dot(xb, ws_ref[...], preferred_element_type=jnp.float32)
    xw_ref[0] = xwl.astype(jnp.bfloat16)
    xw_ref[1] = xwu.astype(jnp.bfloat16)
    sk_ref[...] = sk * (1.0 + 1e-6)
    av = av_ref[...]
    stl = lax.dot_general(av[0:2], xwl, (((1,), (1,)), ((), ())),
                          preferred_element_type=jnp.float32)
    stu = lax.dot_general(av[2:4], xwu, (((1,), (1,)), ((), ())),
                          preferred_element_type=jnp.float32)
    st_ref[0] = stl
    st_ref[1] = stu


_pre = pl.pallas_call(
    _pre_body,
    grid=((N + BN - 1) // BN,),
    in_specs=[
        pl.BlockSpec((BN, D), lambda i: (i, 0)),
        pl.BlockSpec((D, D), lambda i: (0, 0)),
        pl.BlockSpec((D, D), lambda i: (0, 0)),
        pl.BlockSpec((D, D), lambda i: (0, 0)),
        pl.BlockSpec((4, D), lambda i: (0, 0)),
    ],
    out_specs=[
        pl.BlockSpec((2, BN, D), lambda i: (0, i, 0)),
        pl.BlockSpec((BN, D), lambda i: (i, 0)),
        pl.BlockSpec((2, 2, BN), lambda i: (0, 0, i)),
    ],
    out_shape=[
        jax.ShapeDtypeStruct((2, N, D), jnp.bfloat16),
        jax.ShapeDtypeStruct((N, D), jnp.float32),
        jax.ShapeDtypeStruct((2, 2, N), jnp.float32),
    ],
)


def _sc1_body(edges, st, w_out, den_out,
              tgt_v, src_v, s_v, t_v, den_v, w_v):
    c = lax.axis_index("c")
    s = lax.axis_index("s")
    pltpu.sync_copy(st.at[c, 0], s_v)
    pltpu.sync_copy(st.at[c, 1], t_v)

    zv = jnp.zeros((LANES,), jnp.float32)

    @pl.loop(0, N // LANES)
    def _zero_den(i):
        den_v[pl.ds(i * LANES, LANES)] = zv

    @pl.loop(0, NSB)
    def _superchunk(j):
        pltpu.sync_copy(edges.at[c, 0, s, j], tgt_v)
        pltpu.sync_copy(edges.at[c, 1, s, j], src_v)

        @plsc.parallel_loop(0, SBE // LANES, unroll=4)
        def _group(g):
            sl = pl.ds(g * LANES, LANES)
            tg = tgt_v[sl]
            sr = src_v[sl]
            a = plsc.load_gather(s_v, [sr]) + plsc.load_gather(t_v, [tg])
            a = jnp.maximum(a, a * 0.01)
            a = jnp.exp(a)
            plsc.addupdate_scatter(den_v, [tg], a)
            w_v[sl] = a

        pltpu.sync_copy(w_v, w_out.at[c, s, j])

    pltpu.sync_copy(den_v, den_out.at[c, s])


_sc1 = pl.kernel(
    _sc1_body,
    out_type=[
        jax.ShapeDtypeStruct((2, NS, NSB, SBE), jnp.float32),
        jax.ShapeDtypeStruct((2, NS, N), jnp.float32),
    ],
    mesh=plsc.VectorSubcoreMesh(core_axis_name="c", subcore_axis_name="s",
                                num_cores=NC, num_subcores=NS),
    compiler_params=_SC_PARAMS,
    scratch_types=[
        pltpu.VMEM((SBE,), jnp.int32),
        pltpu.VMEM((SBE,), jnp.int32),
        pltpu.VMEM((N,), jnp.float32),
        pltpu.VMEM((N,), jnp.float32),
        pltpu.VMEM((N,), jnp.float32),
        pltpu.VMEM((SBE,), jnp.float32),
    ],
)


def _sc2_body(edges6, w_in, xw, agg_out,
              tgt_sb, src_sb, w_v, rin0, rin1, rin2, rin3, rout0, rout1,
              agg_s, gsem0, gsem1, gsem2, gsem3, ssem0, ssem1):
    c = lax.axis_index("c")
    s = lax.axis_index("s")

    zv = jnp.zeros((LANES,), jnp.float32)

    @pl.loop(0, K)
    def _zero_rows(i):
        for k in range(D // LANES):
            rout0[i, pl.ds(k * LANES, LANES)] = zv

    for j in range(RPT // K):
        pltpu.sync_copy(rout0, agg_s.at[pl.ds(s * RPT + j * K, K)])
    pltpu.sync_copy(rout0.at[pl.ds(0, RPT - (RPT // K) * K)],
                    agg_s.at[pl.ds(s * RPT + (RPT // K) * K,
                                   RPT - (RPT // K) * K)])
    plsc.subcore_barrier()

    xw_c = xw.at[c]
    ins = ((rin0, gsem0), (rin1, gsem1), (rin2, gsem2), (rin3, gsem3))
    outs = ((rout0, ssem0), (rout1, ssem1))

    def _issue_gather(cc, b):
        rows, gsem = ins[b]
        pltpu.async_copy(xw_c.at[src_sb.at[cc]], rows, gsem)

    def _wait_gather(b):
        rows, gsem = ins[b]
        pltpu.make_async_copy(xw_c.at[src_sb.at[0]], rows, gsem).wait()

    def _issue_scatter(cc, b):
        rows, ssem = outs[b]
        pltpu.async_copy(rows, agg_s.at[tgt_sb.at[cc]], ssem, add=True)

    def _wait_scatter(b):
        rows, ssem = outs[b]
        pltpu.make_async_copy(rows, agg_s.at[tgt_sb.at[0]], ssem).wait()

    def _scale(cc, bi, bo):
        rin = ins[bi][0]
        rout = outs[bo][0]
        base = cc * K
        himask = jnp.full((LANES,), -65536, jnp.int32)

        @plsc.parallel_loop(0, K, unroll=8)
        def _edge(e):
            widx = jnp.full((LANES,), base + e, jnp.int32)
            w = plsc.load_gather(w_v, [widx])
            for k in range(D // 32):
                v = plsc.bitcast(rin[e, pl.ds(k * 32, 32)], jnp.int32)
                even = plsc.bitcast(lax.shift_left(v, 16), jnp.float32)
                odd = plsc.bitcast(jnp.bitwise_and(v, himask), jnp.float32)
                rout[e, pl.ds(k * 32, LANES)] = even * w
                rout[e, pl.ds(k * 32 + LANES, LANES)] = odd * w

    @pl.loop(0, NSB)
    def _superchunk(j):
        pltpu.sync_copy(edges6.at[c, 0, s, j], tgt_sb)
        pltpu.sync_copy(edges6.at[c, 1, s, j], src_sb)
        pltpu.sync_copy(w_in.at[c, s, j], w_v)

        for cc in range(GLEAD):
            _issue_gather(cc, cc % NIN)

        def _step(cc, b4, b2):
            @pl.when(cc + GLEAD < SBC)
            def _():
                _issue_gather(cc + GLEAD, (b4 + GLEAD) % NIN)
            _wait_gather(b4)
            @pl.when(cc >= NOUT)
            def _():
                _wait_scatter(b2)
            _scale(cc, b4, b2)
            _issue_scatter(cc, b2)

        @pl.loop(0, SBC // NIN)
        def _quad(p):
            for b in range(NIN):
                _step(p * NIN + b, b, b % NOUT)
        for cc in range((SBC // NIN) * NIN, SBC):
            _step(cc, cc % NIN, cc % NOUT)

        for b in range(NOUT):
            _wait_scatter(b)

    plsc.subcore_barrier()
    pltpu.sync_copy(agg_s.at[pl.ds(s * RPT, RPT)],
                    agg_out.at[c, pl.ds(s * RPT, RPT)])


_sc2 = pl.kernel(
    _sc2_body,
    out_type=jax.ShapeDtypeStruct((2, N, D), jnp.float32),
    mesh=plsc.VectorSubcoreMesh(core_axis_name="c", subcore_axis_name="s",
                                num_cores=NC, num_subcores=NS),
    compiler_params=_SC_PARAMS,
    scratch_types=[
        pltpu.VMEM((SBC, K), jnp.int32),
        pltpu.VMEM((SBC, K), jnp.int32),
        pltpu.VMEM((SBE,), jnp.float32),
        pltpu.VMEM((K, D), jnp.bfloat16),
        pltpu.VMEM((K, D), jnp.bfloat16),
        pltpu.VMEM((K, D), jnp.bfloat16),
        pltpu.VMEM((K, D), jnp.bfloat16),
        pltpu.VMEM((K, D), jnp.float32),
        pltpu.VMEM((K, D), jnp.float32),
        pltpu.VMEM_SHARED((N, D), jnp.float32),
        pltpu.SemaphoreType.DMA,
        pltpu.SemaphoreType.DMA,
        pltpu.SemaphoreType.DMA,
        pltpu.SemaphoreType.DMA,
        pltpu.SemaphoreType.DMA,
        pltpu.SemaphoreType.DMA,
    ],
)


def _post_body(agg_ref, den_ref, sk_ref, out_ref):
    dl = jnp.sum(den_ref[0], axis=0)
    du = jnp.sum(den_ref[1], axis=0)
    rl = 1.0 / (dl + 1e-16)
    ru = 1.0 / (du + 1e-16)
    o = agg_ref[0] * rl[:, None] + agg_ref[1] * ru[:, None] + sk_ref[...]
    out_ref[...] = jnp.maximum(o, 0.0)


_post = pl.pallas_call(
    _post_body,
    grid=((N + BN - 1) // BN,),
    in_specs=[
        pl.BlockSpec((2, BN, D), lambda i: (0, i, 0)),
        pl.BlockSpec((2, NS, BN), lambda i: (0, 0, i)),
        pl.BlockSpec((BN, D), lambda i: (i, 0)),
    ],
    out_specs=pl.BlockSpec((BN, D), lambda i: (i, 0)),
    out_shape=jax.ShapeDtypeStruct((N, D), jnp.float32),
)


@jax.jit
def kernel(x, lower_edge_index, upper_edge_index,
           W_low, a_src_low, a_dst_low,
           W_up, a_src_up, a_dst_up,
           W_skip):
    av = jnp.concatenate([
        a_src_low.reshape(1, D)[:, _PERM], a_dst_low.reshape(1, D)[:, _PERM],
        a_src_up.reshape(1, D)[:, _PERM], a_dst_up.reshape(1, D)[:, _PERM]],
        axis=0)
    xw, sk, st = _pre(x, W_low[:, _PERM], W_up[:, _PERM], W_skip, av)
    edges = jnp.stack([lower_edge_index, upper_edge_index])
    edges5 = edges.reshape(2, 2, NS, NSB, SBE)
    edges6 = edges.reshape(2, 2, NS, NSB, SBC, K)
    w, den = _sc1(edges5, st)
    agg = _sc2(edges6, w, xw)
    return _post(agg, den, sk)


# speedup vs baseline: 28.0387x; 1.0046x over previous
import jax
import jax.numpy as jnp
import numpy as np
from jax import lax
from jax.experimental import pallas as pl
from jax.experimental.pallas import tpu as pltpu
from jax.experimental.pallas import tpu_sc as plsc

N = 10000
E = 320000
D = 128
NC = 2
NS = 16
LANES = 16
K = 80
EPT = E // NS
SBC = 25
SBE = SBC * K
NSB = EPT // SBE
NIN = 4
NOUT = 2
GLEAD = 3
SB1E = 4000
NSB1 = EPT // SB1E
RPT = N // NS
BN = 1024

_SC_PARAMS = pltpu.CompilerParams(use_tc_tiling_on_sc=False,
                                  needs_layout_passes=False)

_PERM = np.empty(D, np.int32)
for _k in range(D // 32):
    for _i in range(16):
        _PERM[32 * _k + 2 * _i] = 32 * _k + _i
        _PERM[32 * _k + 2 * _i + 1] = 32 * _k + 16 + _i


def _pre_body(x_ref, wl_ref, wu_ref, ws_ref, av_ref,
              xw_ref, sk_ref, st_ref):
    xb = x_ref[...]
    xwl = jnp.dot(xb, wl_ref[...], preferred_element_type=jnp.float32)
    xwu = jnp.dot(xb, wu_ref[...], preferred_element_type=jnp.float32)
    sk = jnp.dot(xb, ws_ref[...], preferred_element_type=jnp.float32)
    xw_ref[0] = xwl.astype(jnp.bfloat16)
    xw_ref[1] = xwu.astype(jnp.bfloat16)
    sk_ref[...] = sk * (1.0 + 1e-6)
    av = av_ref[...]
    stl = lax.dot_general(av[0:2], xwl, (((1,), (1,)), ((), ())),
                          preferred_element_type=jnp.float32)
    stu = lax.dot_general(av[2:4], xwu, (((1,), (1,)), ((), ())),
                          preferred_element_type=jnp.float32)
    st_ref[0] = stl
    st_ref[1] = stu


_pre = pl.pallas_call(
    _pre_body,
    grid=((N + BN - 1) // BN,),
    in_specs=[
        pl.BlockSpec((BN, D), lambda i: (i, 0)),
        pl.BlockSpec((D, D), lambda i: (0, 0)),
        pl.BlockSpec((D, D), lambda i: (0, 0)),
        pl.BlockSpec((D, D), lambda i: (0, 0)),
        pl.BlockSpec((4, D), lambda i: (0, 0)),
    ],
    out_specs=[
        pl.BlockSpec((2, BN, D), lambda i: (0, i, 0)),
        pl.BlockSpec((BN, D), lambda i: (i, 0)),
        pl.BlockSpec((2, 2, BN), lambda i: (0, 0, i)),
    ],
    out_shape=[
        jax.ShapeDtypeStruct((2, N, D), jnp.bfloat16),
        jax.ShapeDtypeStruct((N, D), jnp.float32),
        jax.ShapeDtypeStruct((2, 2, N), jnp.float32),
    ],
)


def _sc1_body(edges, st, w_out, den_out,
              tgt_v, src_v, s_v, t_v, den_v, w_v):
    c = lax.axis_index("c")
    s = lax.axis_index("s")
    pltpu.sync_copy(st.at[c, 0], s_v)
    pltpu.sync_copy(st.at[c, 1], t_v)

    zv = jnp.zeros((LANES,), jnp.float32)

    @pl.loop(0, N // LANES)
    def _zero_den(i):
        den_v[pl.ds(i * LANES, LANES)] = zv

    @pl.loop(0, NSB1)
    def _superchunk(j):
        pltpu.sync_copy(edges.at[c, 0, s, j], tgt_v)
        pltpu.sync_copy(edges.at[c, 1, s, j], src_v)

        @plsc.parallel_loop(0, SB1E // LANES, unroll=8)
        def _group(g):
            sl = pl.ds(g * LANES, LANES)
            tg = tgt_v[sl]
            sr = src_v[sl]
            a = plsc.load_gather(s_v, [sr]) + plsc.load_gather(t_v, [tg])
            a = jnp.maximum(a, a * 0.01)
            a = jnp.exp(a)
            plsc.addupdate_scatter(den_v, [tg], a)
            w_v[sl] = a

        pltpu.sync_copy(w_v, w_out.at[c, s, pl.ds(j * SB1E, SB1E)])

    pltpu.sync_copy(den_v, den_out.at[c, s])


_sc1 = pl.kernel(
    _sc1_body,
    out_type=[
        jax.ShapeDtypeStruct((2, NS, EPT), jnp.float32),
        jax.ShapeDtypeStruct((2, NS, N), jnp.float32),
    ],
    mesh=plsc.VectorSubcoreMesh(core_axis_name="c", subcore_axis_name="s",
                                num_cores=NC, num_subcores=NS),
    compiler_params=_SC_PARAMS,
    scratch_types=[
        pltpu.VMEM((SB1E,), jnp.int32),
        pltpu.VMEM((SB1E,), jnp.int32),
        pltpu.VMEM((N,), jnp.float32),
        pltpu.VMEM((N,), jnp.float32),
        pltpu.VMEM((N,), jnp.float32),
        pltpu.VMEM((SB1E,), jnp.float32),
    ],
)


def _sc2_body(edges6, w_in, xw, agg_out,
              tgt_sb, src_sb, w_v, rin0, rin1, rin2, rin3, rout0, rout1,
              agg_s, gsem0, gsem1, gsem2, gsem3, ssem0, ssem1):
    c = lax.axis_index("c")
    s = lax.axis_index("s")

    zv = jnp.zeros((LANES,), jnp.float32)

    @pl.loop(0, K)
    def _zero_rows(i):
        for k in range(D // LANES):
            rout0[i, pl.ds(k * LANES, LANES)] = zv

    for j in range(RPT // K):
        pltpu.sync_copy(rout0, agg_s.at[pl.ds(s * RPT + j * K, K)])
    pltpu.sync_copy(rout0.at[pl.ds(0, RPT - (RPT // K) * K)],
                    agg_s.at[pl.ds(s * RPT + (RPT // K) * K,
                                   RPT - (RPT // K) * K)])
    plsc.subcore_barrier()

    xw_c = xw.at[c]
    ins = ((rin0, gsem0), (rin1, gsem1), (rin2, gsem2), (rin3, gsem3))
    outs = ((rout0, ssem0), (rout1, ssem1))

    def _issue_gather(cc, b):
        rows, gsem = ins[b]
        pltpu.async_copy(xw_c.at[src_sb.at[cc]], rows, gsem)

    def _wait_gather(b):
        rows, gsem = ins[b]
        pltpu.make_async_copy(xw_c.at[src_sb.at[0]], rows, gsem).wait()

    def _issue_scatter(cc, b):
        rows, ssem = outs[b]
        pltpu.async_copy(rows, agg_s.at[tgt_sb.at[cc]], ssem, add=True)

    def _wait_scatter(b):
        rows, ssem = outs[b]
        pltpu.make_async_copy(rows, agg_s.at[tgt_sb.at[0]], ssem).wait()

    def _scale(cc, bi, bo):
        rin = ins[bi][0]
        rout = outs[bo][0]
        base = cc * K
        himask = jnp.full((LANES,), -65536, jnp.int32)

        @plsc.parallel_loop(0, K, unroll=8)
        def _edge(e):
            widx = jnp.full((LANES,), base + e, jnp.int32)
            w = plsc.load_gather(w_v, [widx])
            for k in range(D // 32):
                v = plsc.bitcast(rin[e, pl.ds(k * 32, 32)], jnp.int32)
                even = plsc.bitcast(lax.shift_left(v, 16), jnp.float32)
                odd = plsc.bitcast(jnp.bitwise_and(v, himask), jnp.float32)
                rout[e, pl.ds(k * 32, LANES)] = even * w
                rout[e, pl.ds(k * 32 + LANES, LANES)] = odd * w

    @pl.loop(0, NSB)
    def _superchunk(j):
        pltpu.sync_copy(edges6.at[c, 0, s, j], tgt_sb)
        pltpu.sync_copy(edges6.at[c, 1, s, j], src_sb)
        pltpu.sync_copy(w_in.at[c, s, pl.ds(j * SBE, SBE)], w_v)

        for cc in range(GLEAD):
            _issue_gather(cc, cc % NIN)

        def _step(cc, b4, b2):
            @pl.when(cc + GLEAD < SBC)
            def _():
                _issue_gather(cc + GLEAD, (b4 + GLEAD) % NIN)
            _wait_gather(b4)
            @pl.when(cc >= NOUT)
            def _():
                _wait_scatter(b2)
            _scale(cc, b4, b2)
            _issue_scatter(cc, b2)

        @pl.loop(0, SBC // NIN)
        def _quad(p):
            for b in range(NIN):
                _step(p * NIN + b, b, b % NOUT)
        for cc in range((SBC // NIN) * NIN, SBC):
            _step(cc, cc % NIN, cc % NOUT)

        for b in range(NOUT):
            _wait_scatter(b)

    plsc.subcore_barrier()
    pltpu.sync_copy(agg_s.at[pl.ds(s * RPT, RPT)],
                    agg_out.at[c, pl.ds(s * RPT, RPT)])


_sc2 = pl.kernel(
    _sc2_body,
    out_type=jax.ShapeDtypeStruct((2, N, D), jnp.float32),
    mesh=plsc.VectorSubcoreMesh(core_axis_name="c", subcore_axis_name="s",
                                num_cores=NC, num_subcores=NS),
    compiler_params=_SC_PARAMS,
    scratch_types=[
        pltpu.VMEM((SBC, K), jnp.int32),
        pltpu.VMEM((SBC, K), jnp.int32),
        pltpu.VMEM((SBE,), jnp.float32),
        pltpu.VMEM((K, D), jnp.bfloat16),
        pltpu.VMEM((K, D), jnp.bfloat16),
        pltpu.VMEM((K, D), jnp.bfloat16),
        pltpu.VMEM((K, D), jnp.bfloat16),
        pltpu.VMEM((K, D), jnp.float32),
        pltpu.VMEM((K, D), jnp.float32),
        pltpu.VMEM_SHARED((N, D), jnp.float32),
        pltpu.SemaphoreType.DMA,
        pltpu.SemaphoreType.DMA,
        pltpu.SemaphoreType.DMA,
        pltpu.SemaphoreType.DMA,
        pltpu.SemaphoreType.DMA,
        pltpu.SemaphoreType.DMA,
    ],
)


def _post_body(agg_ref, den_ref, sk_ref, out_ref):
    dl = jnp.sum(den_ref[0], axis=0)
    du = jnp.sum(den_ref[1], axis=0)
    rl = 1.0 / (dl + 1e-16)
    ru = 1.0 / (du + 1e-16)
    o = agg_ref[0] * rl[:, None] + agg_ref[1] * ru[:, None] + sk_ref[...]
    out_ref[...] = jnp.maximum(o, 0.0)


_post = pl.pallas_call(
    _post_body,
    grid=((N + BN - 1) // BN,),
    in_specs=[
        pl.BlockSpec((2, BN, D), lambda i: (0, i, 0)),
        pl.BlockSpec((2, NS, BN), lambda i: (0, 0, i)),
        pl.BlockSpec((BN, D), lambda i: (i, 0)),
    ],
    out_specs=pl.BlockSpec((BN, D), lambda i: (i, 0)),
    out_shape=jax.ShapeDtypeStruct((N, D), jnp.float32),
)


@jax.jit
def kernel(x, lower_edge_index, upper_edge_index,
           W_low, a_src_low, a_dst_low,
           W_up, a_src_up, a_dst_up,
           W_skip):
    av = jnp.concatenate([
        a_src_low.reshape(1, D)[:, _PERM], a_dst_low.reshape(1, D)[:, _PERM],
        a_src_up.reshape(1, D)[:, _PERM], a_dst_up.reshape(1, D)[:, _PERM]],
        axis=0)
    xw, sk, st = _pre(x, W_low[:, _PERM], W_up[:, _PERM], W_skip, av)
    edges = jnp.stack([lower_edge_index, upper_edge_index])
    edges5 = edges.reshape(2, 2, NS, NSB1, SB1E)
    edges6 = edges.reshape(2, 2, NS, NSB, SBC, K)
    w, den = _sc1(edges5, st)
    agg = _sc2(edges6, w, xw)
    return _post(agg, den, sk)


# speedup vs baseline: 31.5869x; 1.1265x over previous
import jax
import jax.numpy as jnp
import numpy as np
from jax import lax
from jax.experimental import pallas as pl
from jax.experimental.pallas import tpu as pltpu
from jax.experimental.pallas import tpu_sc as plsc

N = 10000
E = 320000
D = 128
NC = 2
NS = 16
LANES = 16
K = 80
EPT = E // NS
SBC = 25
SBE = SBC * K
NSB = EPT // SBE
NIN = 4
NOUT = 2
GLEAD = 3
SB1E = 4000
NSB1 = EPT // SB1E
RPT = N // NS
BN = 1024

_SC_PARAMS = pltpu.CompilerParams(use_tc_tiling_on_sc=False,
                                  needs_layout_passes=False)

_PERM = np.empty(D, np.int32)
for _k in range(D // 32):
    for _i in range(16):
        _PERM[32 * _k + 2 * _i] = 32 * _k + _i
        _PERM[32 * _k + 2 * _i + 1] = 32 * _k + 16 + _i


def _pre_body(x_ref, wl_ref, wu_ref, ws_ref, av_ref,
              xw_ref, sk_ref, st_ref):
    xb = x_ref[...]
    xwl = jnp.dot(xb, wl_ref[...], preferred_element_type=jnp.float32)
    xwu = jnp.dot(xb, wu_ref[...], preferred_element_type=jnp.float32)
    sk = jnp.dot(xb, ws_ref[...], preferred_element_type=jnp.float32)
    xw_ref[0] = xwl.astype(jnp.bfloat16)
    xw_ref[1] = xwu.astype(jnp.bfloat16)
    sk_ref[...] = sk * (1.0 + 1e-6)
    av = av_ref[...]
    stl = lax.dot_general(av[0:2], xwl, (((1,), (1,)), ((), ())),
                          preferred_element_type=jnp.float32)
    stu = lax.dot_general(av[2:4], xwu, (((1,), (1,)), ((), ())),
                          preferred_element_type=jnp.float32)
    st_ref[0] = stl
    st_ref[1] = stu


_pre = pl.pallas_call(
    _pre_body,
    grid=((N + BN - 1) // BN,),
    in_specs=[
        pl.BlockSpec((BN, D), lambda i: (i, 0)),
        pl.BlockSpec((D, D), lambda i: (0, 0)),
        pl.BlockSpec((D, D), lambda i: (0, 0)),
        pl.BlockSpec((D, D), lambda i: (0, 0)),
        pl.BlockSpec((4, D), lambda i: (0, 0)),
    ],
    out_specs=[
        pl.BlockSpec((2, BN, D), lambda i: (0, i, 0)),
        pl.BlockSpec((BN, D), lambda i: (i, 0)),
        pl.BlockSpec((2, 2, BN), lambda i: (0, 0, i)),
    ],
    out_shape=[
        jax.ShapeDtypeStruct((2, N, D), jnp.bfloat16),
        jax.ShapeDtypeStruct((N, D), jnp.float32),
        jax.ShapeDtypeStruct((2, 2, N), jnp.float32),
    ],
)


def _sc1_body(elow, eup, st, w_out, den_out,
              tgt_v, src_v, s_v, t_v, den_v, w_v):
    c = lax.axis_index("c")
    s = lax.axis_index("s")
    pltpu.sync_copy(st.at[c, 0], s_v)
    pltpu.sync_copy(st.at[c, 1], t_v)

    zv = jnp.zeros((LANES,), jnp.float32)

    @pl.loop(0, N // LANES)
    def _zero_den(i):
        den_v[pl.ds(i * LANES, LANES)] = zv

    @pl.loop(0, NSB1)
    def _superchunk(j):
        @pl.when(c == 0)
        def _():
            pltpu.sync_copy(elow.at[0, s, j], tgt_v)
            pltpu.sync_copy(elow.at[1, s, j], src_v)
        @pl.when(c == 1)
        def _():
            pltpu.sync_copy(eup.at[0, s, j], tgt_v)
            pltpu.sync_copy(eup.at[1, s, j], src_v)

        @plsc.parallel_loop(0, SB1E // LANES, unroll=8)
        def _group(g):
            sl = pl.ds(g * LANES, LANES)
            tg = tgt_v[sl]
            sr = src_v[sl]
            a = plsc.load_gather(s_v, [sr]) + plsc.load_gather(t_v, [tg])
            a = jnp.maximum(a, a * 0.01)
            a = jnp.exp(a)
            plsc.addupdate_scatter(den_v, [tg], a)
            w_v[sl] = a

        pltpu.sync_copy(w_v, w_out.at[c, s, pl.ds(j * SB1E, SB1E)])

    pltpu.sync_copy(den_v, den_out.at[c, s])


_sc1 = pl.kernel(
    _sc1_body,
    out_type=[
        jax.ShapeDtypeStruct((2, NS, EPT), jnp.float32),
        jax.ShapeDtypeStruct((2, NS, N), jnp.float32),
    ],
    mesh=plsc.VectorSubcoreMesh(core_axis_name="c", subcore_axis_name="s",
                                num_cores=NC, num_subcores=NS),
    compiler_params=_SC_PARAMS,
    scratch_types=[
        pltpu.VMEM((SB1E,), jnp.int32),
        pltpu.VMEM((SB1E,), jnp.int32),
        pltpu.VMEM((N,), jnp.float32),
        pltpu.VMEM((N,), jnp.float32),
        pltpu.VMEM((N,), jnp.float32),
        pltpu.VMEM((SB1E,), jnp.float32),
    ],
)


def _sc2_body(elow, eup, w_in, xw, agg_out,
              tgt_sb, src_sb, w_v, rin0, rin1, rin2, rin3, rout0, rout1,
              agg_s, gsem0, gsem1, gsem2, gsem3, ssem0, ssem1):
    c = lax.axis_index("c")
    s = lax.axis_index("s")

    zv = jnp.zeros((LANES,), jnp.float32)

    @pl.loop(0, K)
    def _zero_rows(i):
        for k in range(D // LANES):
            rout0[i, pl.ds(k * LANES, LANES)] = zv

    for j in range(RPT // K):
        pltpu.sync_copy(rout0, agg_s.at[pl.ds(s * RPT + j * K, K)])
    pltpu.sync_copy(rout0.at[pl.ds(0, RPT - (RPT // K) * K)],
                    agg_s.at[pl.ds(s * RPT + (RPT // K) * K,
                                   RPT - (RPT // K) * K)])
    plsc.subcore_barrier()

    xw_c = xw.at[c]
    ins = ((rin0, gsem0), (rin1, gsem1), (rin2, gsem2), (rin3, gsem3))
    outs = ((rout0, ssem0), (rout1, ssem1))

    def _issue_gather(cc, b):
        rows, gsem = ins[b]
        pltpu.async_copy(xw_c.at[src_sb.at[cc]], rows, gsem)

    def _wait_gather(b):
        rows, gsem = ins[b]
        pltpu.make_async_copy(xw_c.at[src_sb.at[0]], rows, gsem).wait()

    def _issue_scatter(cc, b):
        rows, ssem = outs[b]
        pltpu.async_copy(rows, agg_s.at[tgt_sb.at[cc]], ssem, add=True)

    def _wait_scatter(b):
        rows, ssem = outs[b]
        pltpu.make_async_copy(rows, agg_s.at[tgt_sb.at[0]], ssem).wait()

    def _scale(cc, bi, bo):
        rin = ins[bi][0]
        rout = outs[bo][0]
        base = cc * K
        himask = jnp.full((LANES,), -65536, jnp.int32)

        @plsc.parallel_loop(0, K, unroll=8)
        def _edge(e):
            widx = jnp.full((LANES,), base + e, jnp.int32)
            w = plsc.load_gather(w_v, [widx])
            for k in range(D // 32):
                v = plsc.bitcast(rin[e, pl.ds(k * 32, 32)], jnp.int32)
                even = plsc.bitcast(lax.shift_left(v, 16), jnp.float32)
                odd = plsc.bitcast(jnp.bitwise_and(v, himask), jnp.float32)
                rout[e, pl.ds(k * 32, LANES)] = even * w
                rout[e, pl.ds(k * 32 + LANES, LANES)] = odd * w

    @pl.loop(0, NSB)
    def _superchunk(j):
        @pl.when(c == 0)
        def _():
            pltpu.sync_copy(elow.at[0, s, j], tgt_sb)
            pltpu.sync_copy(elow.at[1, s, j], src_sb)
        @pl.when(c == 1)
        def _():
            pltpu.sync_copy(eup.at[0, s, j], tgt_sb)
            pltpu.sync_copy(eup.at[1, s, j], src_sb)
        pltpu.sync_copy(w_in.at[c, s, pl.ds(j * SBE, SBE)], w_v)

        for cc in range(GLEAD):
            _issue_gather(cc, cc % NIN)

        def _step(cc, b4, b2):
            @pl.when(cc + GLEAD < SBC)
            def _():
                _issue_gather(cc + GLEAD, (b4 + GLEAD) % NIN)
            _wait_gather(b4)
            @pl.when(cc >= NOUT)
            def _():
                _wait_scatter(b2)
            _scale(cc, b4, b2)
            _issue_scatter(cc, b2)

        @pl.loop(0, SBC // NIN)
        def _quad(p):
            for b in range(NIN):
                _step(p * NIN + b, b, b % NOUT)
        for cc in range((SBC // NIN) * NIN, SBC):
            _step(cc, cc % NIN, cc % NOUT)

        for b in range(NOUT):
            _wait_scatter(b)

    plsc.subcore_barrier()
    pltpu.sync_copy(agg_s.at[pl.ds(s * RPT, RPT)],
                    agg_out.at[c, pl.ds(s * RPT, RPT)])


_sc2 = pl.kernel(
    _sc2_body,
    out_type=jax.ShapeDtypeStruct((2, N, D), jnp.float32),
    mesh=plsc.VectorSubcoreMesh(core_axis_name="c", subcore_axis_name="s",
                                num_cores=NC, num_subcores=NS),
    compiler_params=_SC_PARAMS,
    scratch_types=[
        pltpu.VMEM((SBC, K), jnp.int32),
        pltpu.VMEM((SBC, K), jnp.int32),
        pltpu.VMEM((SBE,), jnp.float32),
        pltpu.VMEM((K, D), jnp.bfloat16),
        pltpu.VMEM((K, D), jnp.bfloat16),
        pltpu.VMEM((K, D), jnp.bfloat16),
        pltpu.VMEM((K, D), jnp.bfloat16),
        pltpu.VMEM((K, D), jnp.float32),
        pltpu.VMEM((K, D), jnp.float32),
        pltpu.VMEM_SHARED((N, D), jnp.float32),
        pltpu.SemaphoreType.DMA,
        pltpu.SemaphoreType.DMA,
        pltpu.SemaphoreType.DMA,
        pltpu.SemaphoreType.DMA,
        pltpu.SemaphoreType.DMA,
        pltpu.SemaphoreType.DMA,
    ],
)


def _post_body(agg_ref, den_ref, sk_ref, out_ref):
    dl = jnp.sum(den_ref[0], axis=0)
    du = jnp.sum(den_ref[1], axis=0)
    rl = 1.0 / (dl + 1e-16)
    ru = 1.0 / (du + 1e-16)
    o = agg_ref[0] * rl[:, None] + agg_ref[1] * ru[:, None] + sk_ref[...]
    out_ref[...] = jnp.maximum(o, 0.0)


_post = pl.pallas_call(
    _post_body,
    grid=((N + BN - 1) // BN,),
    in_specs=[
        pl.BlockSpec((2, BN, D), lambda i: (0, i, 0)),
        pl.BlockSpec((2, NS, BN), lambda i: (0, 0, i)),
        pl.BlockSpec((BN, D), lambda i: (i, 0)),
    ],
    out_specs=pl.BlockSpec((BN, D), lambda i: (i, 0)),
    out_shape=jax.ShapeDtypeStruct((N, D), jnp.float32),
)


@jax.jit
def kernel(x, lower_edge_index, upper_edge_index,
           W_low, a_src_low, a_dst_low,
           W_up, a_src_up, a_dst_up,
           W_skip):
    av = jnp.concatenate([
        a_src_low.reshape(1, D)[:, _PERM], a_dst_low.reshape(1, D)[:, _PERM],
        a_src_up.reshape(1, D)[:, _PERM], a_dst_up.reshape(1, D)[:, _PERM]],
        axis=0)
    xw, sk, st = _pre(x, W_low[:, _PERM], W_up[:, _PERM], W_skip, av)
    elow5 = lower_edge_index.reshape(2, NS, NSB1, SB1E)
    eup5 = upper_edge_index.reshape(2, NS, NSB1, SB1E)
    elow6 = lower_edge_index.reshape(2, NS, NSB, SBC, K)
    eup6 = upper_edge_index.reshape(2, NS, NSB, SBC, K)
    w, den = _sc1(elow5, eup5, st)
    agg = _sc2(elow6, eup6, w, xw)
    return _post(agg, den, sk)


# speedup vs baseline: 32.4711x; 1.0280x over previous
import jax
import jax.numpy as jnp
import numpy as np
from jax import lax
from jax.experimental import pallas as pl
from jax.experimental.pallas import tpu as pltpu
from jax.experimental.pallas import tpu_sc as plsc

N = 10000
E = 320000
D = 128
NC = 2
NS = 16
LANES = 16
K = 80
EPT = E // NS
SBC = 50
SBE = SBC * K
NSB = EPT // SBE
NIN = 3
NOUT = 2
GLEAD = 2
NUR = 6
SB1E = 4000
NSB1 = EPT // SB1E
RPT = N // NS
BN = 1024

_SC_PARAMS = pltpu.CompilerParams(use_tc_tiling_on_sc=False,
                                  needs_layout_passes=False)

_PERM = np.empty(D, np.int32)
for _k in range(D // 32):
    for _i in range(16):
        _PERM[32 * _k + 2 * _i] = 32 * _k + _i
        _PERM[32 * _k + 2 * _i + 1] = 32 * _k + 16 + _i


def _pre_body(x_ref, wl_ref, wu_ref, ws_ref, av_ref,
              xw_ref, sk_ref, st_ref):
    xb = x_ref[...]
    xwl = jnp.dot(xb, wl_ref[...], preferred_element_type=jnp.float32)
    xwu = jnp.dot(xb, wu_ref[...], preferred_element_type=jnp.float32)
    sk = jnp.dot(xb, ws_ref[...], preferred_element_type=jnp.float32)
    xw_ref[0] = xwl.astype(jnp.bfloat16)
    xw_ref[1] = xwu.astype(jnp.bfloat16)
    sk_ref[...] = sk * (1.0 + 1e-6)
    av = av_ref[...]
    stl = lax.dot_general(av[0:2], xwl, (((1,), (1,)), ((), ())),
                          preferred_element_type=jnp.float32)
    stu = lax.dot_general(av[2:4], xwu, (((1,), (1,)), ((), ())),
                          preferred_element_type=jnp.float32)
    st_ref[0] = stl
    st_ref[1] = stu


_pre = pl.pallas_call(
    _pre_body,
    grid=((N + BN - 1) // BN,),
    in_specs=[
        pl.BlockSpec((BN, D), lambda i: (i, 0)),
        pl.BlockSpec((D, D), lambda i: (0, 0)),
        pl.BlockSpec((D, D), lambda i: (0, 0)),
        pl.BlockSpec((D, D), lambda i: (0, 0)),
        pl.BlockSpec((4, D), lambda i: (0, 0)),
    ],
    out_specs=[
        pl.BlockSpec((2, BN, D), lambda i: (0, i, 0)),
        pl.BlockSpec((BN, D), lambda i: (i, 0)),
        pl.BlockSpec((2, 2, BN), lambda i: (0, 0, i)),
    ],
    out_shape=[
        jax.ShapeDtypeStruct((2, N, D), jnp.bfloat16),
        jax.ShapeDtypeStruct((N, D), jnp.float32),
        jax.ShapeDtypeStruct((2, 2, N), jnp.float32),
    ],
)


def _sc1_body(elow, eup, st, w_out, den_out,
              tgt_v, src_v, s_v, t_v, den_v, w_v):
    c = lax.axis_index("c")
    s = lax.axis_index("s")
    pltpu.sync_copy(st.at[c, 0], s_v)
    pltpu.sync_copy(st.at[c, 1], t_v)

    zv = jnp.zeros((LANES,), jnp.float32)

    @pl.loop(0, N // LANES)
    def _zero_den(i):
        den_v[pl.ds(i * LANES, LANES)] = zv

    @pl.loop(0, NSB1)
    def _superchunk(j):
        @pl.when(c == 0)
        def _():
            pltpu.sync_copy(elow.at[0, s, j], tgt_v)
            pltpu.sync_copy(elow.at[1, s, j], src_v)
        @pl.when(c == 1)
        def _():
            pltpu.sync_copy(eup.at[0, s, j], tgt_v)
            pltpu.sync_copy(eup.at[1, s, j], src_v)

        @plsc.parallel_loop(0, SB1E // LANES, unroll=8)
        def _group(g):
            sl = pl.ds(g * LANES, LANES)
            tg = tgt_v[sl]
            sr = src_v[sl]
            a = plsc.load_gather(s_v, [sr]) + plsc.load_gather(t_v, [tg])
            a = jnp.maximum(a, a * 0.01)
            a = jnp.exp(a)
            plsc.addupdate_scatter(den_v, [tg], a)
            w_v[sl] = a

        pltpu.sync_copy(w_v, w_out.at[c, s, pl.ds(j * SB1E, SB1E)])

    pltpu.sync_copy(den_v, den_out.at[c, s])


_sc1 = pl.kernel(
    _sc1_body,
    out_type=[
        jax.ShapeDtypeStruct((2, NS, EPT), jnp.float32),
        jax.ShapeDtypeStruct((2, NS, N), jnp.float32),
    ],
    mesh=plsc.VectorSubcoreMesh(core_axis_name="c", subcore_axis_name="s",
                                num_cores=NC, num_subcores=NS),
    compiler_params=_SC_PARAMS,
    scratch_types=[
        pltpu.VMEM((SB1E,), jnp.int32),
        pltpu.VMEM((SB1E,), jnp.int32),
        pltpu.VMEM((N,), jnp.float32),
        pltpu.VMEM((N,), jnp.float32),
        pltpu.VMEM((N,), jnp.float32),
        pltpu.VMEM((SB1E,), jnp.float32),
    ],
)


def _sc2_body(elow, eup, w_in, xw, agg_out,
              tgt_sb, src_sb, w_v, rin0, rin1, rin2, rout0, rout1,
              agg_s, gsem0, gsem1, gsem2, ssem0, ssem1):
    c = lax.axis_index("c")
    s = lax.axis_index("s")

    zv = jnp.zeros((LANES,), jnp.float32)

    @pl.loop(0, K)
    def _zero_rows(i):
        for k in range(D // LANES):
            rout0[i, pl.ds(k * LANES, LANES)] = zv

    for j in range(RPT // K):
        pltpu.sync_copy(rout0, agg_s.at[pl.ds(s * RPT + j * K, K)])
    pltpu.sync_copy(rout0.at[pl.ds(0, RPT - (RPT // K) * K)],
                    agg_s.at[pl.ds(s * RPT + (RPT // K) * K,
                                   RPT - (RPT // K) * K)])
    plsc.subcore_barrier()

    xw_c = xw.at[c]
    ins = ((rin0, gsem0), (rin1, gsem1), (rin2, gsem2))
    outs = ((rout0, ssem0), (rout1, ssem1))

    def _issue_gather(cc, b):
        rows, gsem = ins[b]
        pltpu.async_copy(xw_c.at[src_sb.at[cc]], rows, gsem)

    def _wait_gather(b):
        rows, gsem = ins[b]
        pltpu.make_async_copy(xw_c.at[src_sb.at[0]], rows, gsem).wait()

    def _issue_scatter(cc, b):
        rows, ssem = outs[b]
        pltpu.async_copy(rows, agg_s.at[tgt_sb.at[cc]], ssem, add=True)

    def _wait_scatter(b):
        rows, ssem = outs[b]
        pltpu.make_async_copy(rows, agg_s.at[tgt_sb.at[0]], ssem).wait()

    def _scale(cc, bi, bo):
        rin = ins[bi][0]
        rout = outs[bo][0]
        base = cc * K
        himask = jnp.full((LANES,), -65536, jnp.int32)

        @plsc.parallel_loop(0, K, unroll=8)
        def _edge(e):
            widx = jnp.full((LANES,), base + e, jnp.int32)
            w = plsc.load_gather(w_v, [widx])
            for k in range(D // 32):
                v = plsc.bitcast(rin[e, pl.ds(k * 32, 32)], jnp.int32)
                even = plsc.bitcast(lax.shift_left(v, 16), jnp.float32)
                odd = plsc.bitcast(jnp.bitwise_and(v, himask), jnp.float32)
                rout[e, pl.ds(k * 32, LANES)] = even * w
                rout[e, pl.ds(k * 32 + LANES, LANES)] = odd * w

    @pl.loop(0, NSB)
    def _superchunk(j):
        @pl.when(c == 0)
        def _():
            pltpu.sync_copy(elow.at[0, s, j], tgt_sb)
            pltpu.sync_copy(elow.at[1, s, j], src_sb)
        @pl.when(c == 1)
        def _():
            pltpu.sync_copy(eup.at[0, s, j], tgt_sb)
            pltpu.sync_copy(eup.at[1, s, j], src_sb)
        pltpu.sync_copy(w_in.at[c, s, pl.ds(j * SBE, SBE)], w_v)

        for cc in range(GLEAD):
            _issue_gather(cc, cc % NIN)

        def _step(cc, b4, b2):
            @pl.when(cc + GLEAD < SBC)
            def _():
                _issue_gather(cc + GLEAD, (b4 + GLEAD) % NIN)
            _wait_gather(b4)
            @pl.when(cc >= NOUT)
            def _():
                _wait_scatter(b2)
            _scale(cc, b4, b2)
            _issue_scatter(cc, b2)

        @pl.loop(0, SBC // NUR)
        def _six(p):
            for b in range(NUR):
                _step(p * NUR + b, b % NIN, b % NOUT)
        for cc in range((SBC // NUR) * NUR, SBC):
            _step(cc, cc % NIN, cc % NOUT)

        for b in range(NOUT):
            _wait_scatter(b)

    plsc.subcore_barrier()
    pltpu.sync_copy(agg_s.at[pl.ds(s * RPT, RPT)],
                    agg_out.at[c, pl.ds(s * RPT, RPT)])


_sc2 = pl.kernel(
    _sc2_body,
    out_type=jax.ShapeDtypeStruct((2, N, D), jnp.float32),
    mesh=plsc.VectorSubcoreMesh(core_axis_name="c", subcore_axis_name="s",
                                num_cores=NC, num_subcores=NS),
    compiler_params=_SC_PARAMS,
    scratch_types=[
        pltpu.VMEM((SBC, K), jnp.int32),
        pltpu.VMEM((SBC, K), jnp.int32),
        pltpu.VMEM((SBE,), jnp.float32),
        pltpu.VMEM((K, D), jnp.bfloat16),
        pltpu.VMEM((K, D), jnp.bfloat16),
        pltpu.VMEM((K, D), jnp.bfloat16),
        pltpu.VMEM((K, D), jnp.float32),
        pltpu.VMEM((K, D), jnp.float32),
        pltpu.VMEM_SHARED((N, D), jnp.float32),
        pltpu.SemaphoreType.DMA,
        pltpu.SemaphoreType.DMA,
        pltpu.SemaphoreType.DMA,
        pltpu.SemaphoreType.DMA,
        pltpu.SemaphoreType.DMA,
    ],
)


def _post_body(agg_ref, den_ref, sk_ref, out_ref):
    dl = jnp.sum(den_ref[0], axis=0)
    du = jnp.sum(den_ref[1], axis=0)
    rl = 1.0 / (dl + 1e-16)
    ru = 1.0 / (du + 1e-16)
    o = agg_ref[0] * rl[:, None] + agg_ref[1] * ru[:, None] + sk_ref[...]
    out_ref[...] = jnp.maximum(o, 0.0)


_post = pl.pallas_call(
    _post_body,
    grid=((N + BN - 1) // BN,),
    in_specs=[
        pl.BlockSpec((2, BN, D), lambda i: (0, i, 0)),
        pl.BlockSpec((2, NS, BN), lambda i: (0, 0, i)),
        pl.BlockSpec((BN, D), lambda i: (i, 0)),
    ],
    out_specs=pl.BlockSpec((BN, D), lambda i: (i, 0)),
    out_shape=jax.ShapeDtypeStruct((N, D), jnp.float32),
)


@jax.jit
def kernel(x, lower_edge_index, upper_edge_index,
           W_low, a_src_low, a_dst_low,
           W_up, a_src_up, a_dst_up,
           W_skip):
    av = jnp.concatenate([
        a_src_low.reshape(1, D)[:, _PERM], a_dst_low.reshape(1, D)[:, _PERM],
        a_src_up.reshape(1, D)[:, _PERM], a_dst_up.reshape(1, D)[:, _PERM]],
        axis=0)
    xw, sk, st = _pre(x, W_low[:, _PERM], W_up[:, _PERM], W_skip, av)
    elow5 = lower_edge_index.reshape(2, NS, NSB1, SB1E)
    eup5 = upper_edge_index.reshape(2, NS, NSB1, SB1E)
    elow6 = lower_edge_index.reshape(2, NS, NSB, SBC, K)
    eup6 = upper_edge_index.reshape(2, NS, NSB, SBC, K)
    w, den = _sc1(elow5, eup5, st)
    agg = _sc2(elow6, eup6, w, xw)
    return _post(agg, den, sk)


# speedup vs baseline: 33.1376x; 1.0205x over previous
import jax
import jax.numpy as jnp
import numpy as np
from jax import lax
from jax.experimental import pallas as pl
from jax.experimental.pallas import tpu as pltpu
from jax.experimental.pallas import tpu_sc as plsc

N = 10000
E = 320000
D = 128
NC = 2
NS = 16
LANES = 16
K = 80
EPT = E // NS
SBC = 50
SBE = SBC * K
NSB = EPT // SBE
NIN = 3
NOUT = 2
GLEAD = 2
NUR = 6
SB1E = 4000
NSB1 = EPT // SB1E
RPT = N // NS
BN = 2560

_SC_PARAMS = pltpu.CompilerParams(use_tc_tiling_on_sc=False,
                                  needs_layout_passes=False)

_PERM = np.empty(D, np.int32)
for _k in range(D // 32):
    for _i in range(16):
        _PERM[32 * _k + 2 * _i] = 32 * _k + _i
        _PERM[32 * _k + 2 * _i + 1] = 32 * _k + 16 + _i


def _pre_body(x_ref, wl_ref, wu_ref, ws_ref, av_ref,
              xw_ref, sk_ref, st_ref):
    xb = x_ref[...]
    xwl = jnp.dot(xb, wl_ref[...], preferred_element_type=jnp.float32)
    xwu = jnp.dot(xb, wu_ref[...], preferred_element_type=jnp.float32)
    sk = jnp.dot(xb, ws_ref[...], preferred_element_type=jnp.float32)
    xw_ref[0] = xwl.astype(jnp.bfloat16)
    xw_ref[1] = xwu.astype(jnp.bfloat16)
    sk_ref[...] = sk * (1.0 + 1e-6)
    av = av_ref[...]
    stl = lax.dot_general(av[0:2], xwl, (((1,), (1,)), ((), ())),
                          preferred_element_type=jnp.float32)
    stu = lax.dot_general(av[2:4], xwu, (((1,), (1,)), ((), ())),
                          preferred_element_type=jnp.float32)
    st_ref[0] = stl
    st_ref[1] = stu


_pre = pl.pallas_call(
    _pre_body,
    grid=((N + BN - 1) // BN,),
    in_specs=[
        pl.BlockSpec((BN, D), lambda i: (i, 0)),
        pl.BlockSpec((D, D), lambda i: (0, 0)),
        pl.BlockSpec((D, D), lambda i: (0, 0)),
        pl.BlockSpec((D, D), lambda i: (0, 0)),
        pl.BlockSpec((4, D), lambda i: (0, 0)),
    ],
    out_specs=[
        pl.BlockSpec((2, BN, D), lambda i: (0, i, 0)),
        pl.BlockSpec((BN, D), lambda i: (i, 0)),
        pl.BlockSpec((2, 2, BN), lambda i: (0, 0, i)),
    ],
    out_shape=[
        jax.ShapeDtypeStruct((2, N, D), jnp.bfloat16),
        jax.ShapeDtypeStruct((N, D), jnp.float32),
        jax.ShapeDtypeStruct((2, 2, N), jnp.float32),
    ],
)


def _sc1_body(elow, eup, st, w_out, den_out,
              tgt_v, src_v, s_v, t_v, den_v, w_v):
    c = lax.axis_index("c")
    s = lax.axis_index("s")
    pltpu.sync_copy(st.at[c, 0], s_v)
    pltpu.sync_copy(st.at[c, 1], t_v)

    zv = jnp.zeros((LANES,), jnp.float32)

    @pl.loop(0, N // LANES)
    def _zero_den(i):
        den_v[pl.ds(i * LANES, LANES)] = zv

    @pl.loop(0, NSB1)
    def _superchunk(j):
        @pl.when(c == 0)
        def _():
            pltpu.sync_copy(elow.at[0, s, j], tgt_v)
            pltpu.sync_copy(elow.at[1, s, j], src_v)
        @pl.when(c == 1)
        def _():
            pltpu.sync_copy(eup.at[0, s, j], tgt_v)
            pltpu.sync_copy(eup.at[1, s, j], src_v)

        @plsc.parallel_loop(0, SB1E // LANES, unroll=8)
        def _group(g):
            sl = pl.ds(g * LANES, LANES)
            tg = tgt_v[sl]
            sr = src_v[sl]
            a = plsc.load_gather(s_v, [sr]) + plsc.load_gather(t_v, [tg])
            a = jnp.maximum(a, a * 0.01)
            a = jnp.exp(a)
            plsc.addupdate_scatter(den_v, [tg], a)
            w_v[sl] = a

        pltpu.sync_copy(w_v, w_out.at[c, s, pl.ds(j * SB1E, SB1E)])

    pltpu.sync_copy(den_v, den_out.at[c, s])


_sc1 = pl.kernel(
    _sc1_body,
    out_type=[
        jax.ShapeDtypeStruct((2, NS, EPT), jnp.float32),
        jax.ShapeDtypeStruct((2, NS, N), jnp.float32),
    ],
    mesh=plsc.VectorSubcoreMesh(core_axis_name="c", subcore_axis_name="s",
                                num_cores=NC, num_subcores=NS),
    compiler_params=_SC_PARAMS,
    scratch_types=[
        pltpu.VMEM((SB1E,), jnp.int32),
        pltpu.VMEM((SB1E,), jnp.int32),
        pltpu.VMEM((N,), jnp.float32),
        pltpu.VMEM((N,), jnp.float32),
        pltpu.VMEM((N,), jnp.float32),
        pltpu.VMEM((SB1E,), jnp.float32),
    ],
)


def _sc2_body(elow, eup, w_in, xw, agg_out,
              tgt_sb, src_sb, w_v, rin0, rin1, rin2, rout0, rout1,
              agg_s, gsem0, gsem1, gsem2, ssem0, ssem1):
    c = lax.axis_index("c")
    s = lax.axis_index("s")

    zv = jnp.zeros((LANES,), jnp.float32)

    @pl.loop(0, K)
    def _zero_rows(i):
        for k in range(D // LANES):
            rout0[i, pl.ds(k * LANES, LANES)] = zv

    for j in range(RPT // K):
        pltpu.sync_copy(rout0, agg_s.at[pl.ds(s * RPT + j * K, K)])
    pltpu.sync_copy(rout0.at[pl.ds(0, RPT - (RPT // K) * K)],
                    agg_s.at[pl.ds(s * RPT + (RPT // K) * K,
                                   RPT - (RPT // K) * K)])
    plsc.subcore_barrier()

    xw_c = xw.at[c]
    ins = ((rin0, gsem0), (rin1, gsem1), (rin2, gsem2))
    outs = ((rout0, ssem0), (rout1, ssem1))

    def _issue_gather(cc, b):
        rows, gsem = ins[b]
        pltpu.async_copy(xw_c.at[src_sb.at[cc]], rows, gsem)

    def _wait_gather(b):
        rows, gsem = ins[b]
        pltpu.make_async_copy(xw_c.at[src_sb.at[0]], rows, gsem).wait()

    def _issue_scatter(cc, b):
        rows, ssem = outs[b]
        pltpu.async_copy(rows, agg_s.at[tgt_sb.at[cc]], ssem, add=True)

    def _wait_scatter(b):
        rows, ssem = outs[b]
        pltpu.make_async_copy(rows, agg_s.at[tgt_sb.at[0]], ssem).wait()

    def _scale(cc, bi, bo):
        rin = ins[bi][0]
        rout = outs[bo][0]
        base = cc * K
        himask = jnp.full((LANES,), -65536, jnp.int32)

        @plsc.parallel_loop(0, K, unroll=8)
        def _edge(e):
            widx = jnp.full((LANES,), base + e, jnp.int32)
            w = plsc.load_gather(w_v, [widx])
            for k in range(D // 32):
                v = plsc.bitcast(rin[e, pl.ds(k * 32, 32)], jnp.int32)
                even = plsc.bitcast(lax.shift_left(v, 16), jnp.float32)
                odd = plsc.bitcast(jnp.bitwise_and(v, himask), jnp.float32)
                rout[e, pl.ds(k * 32, LANES)] = even * w
                rout[e, pl.ds(k * 32 + LANES, LANES)] = odd * w

    @pl.loop(0, NSB)
    def _superchunk(j):
        @pl.when(c == 0)
        def _():
            pltpu.sync_copy(elow.at[0, s, j], tgt_sb)
            pltpu.sync_copy(elow.at[1, s, j], src_sb)
        @pl.when(c == 1)
        def _():
            pltpu.sync_copy(eup.at[0, s, j], tgt_sb)
            pltpu.sync_copy(eup.at[1, s, j], src_sb)
        pltpu.sync_copy(w_in.at[c, s, pl.ds(j * SBE, SBE)], w_v)

        for cc in range(GLEAD):
            _issue_gather(cc, cc % NIN)

        def _step(cc, b4, b2):
            @pl.when(cc + GLEAD < SBC)
            def _():
                _issue_gather(cc + GLEAD, (b4 + GLEAD) % NIN)
            _wait_gather(b4)
            @pl.when(cc >= NOUT)
            def _():
                _wait_scatter(b2)
            _scale(cc, b4, b2)
            _issue_scatter(cc, b2)

        @pl.loop(0, SBC // NUR)
        def _six(p):
            for b in range(NUR):
                _step(p * NUR + b, b % NIN, b % NOUT)
        for cc in range((SBC // NUR) * NUR, SBC):
            _step(cc, cc % NIN, cc % NOUT)

        for b in range(NOUT):
            _wait_scatter(b)

    plsc.subcore_barrier()
    pltpu.sync_copy(agg_s.at[pl.ds(s * RPT, RPT)],
                    agg_out.at[c, pl.ds(s * RPT, RPT)])


_sc2 = pl.kernel(
    _sc2_body,
    out_type=jax.ShapeDtypeStruct((2, N, D), jnp.float32),
    mesh=plsc.VectorSubcoreMesh(core_axis_name="c", subcore_axis_name="s",
                                num_cores=NC, num_subcores=NS),
    compiler_params=_SC_PARAMS,
    scratch_types=[
        pltpu.VMEM((SBC, K), jnp.int32),
        pltpu.VMEM((SBC, K), jnp.int32),
        pltpu.VMEM((SBE,), jnp.float32),
        pltpu.VMEM((K, D), jnp.bfloat16),
        pltpu.VMEM((K, D), jnp.bfloat16),
        pltpu.VMEM((K, D), jnp.bfloat16),
        pltpu.VMEM((K, D), jnp.float32),
        pltpu.VMEM((K, D), jnp.float32),
        pltpu.VMEM_SHARED((N, D), jnp.float32),
        pltpu.SemaphoreType.DMA,
        pltpu.SemaphoreType.DMA,
        pltpu.SemaphoreType.DMA,
        pltpu.SemaphoreType.DMA,
        pltpu.SemaphoreType.DMA,
    ],
)


def _post_body(agg_ref, den_ref, sk_ref, out_ref):
    dl = jnp.sum(den_ref[0], axis=0)
    du = jnp.sum(den_ref[1], axis=0)
    rl = 1.0 / (dl + 1e-16)
    ru = 1.0 / (du + 1e-16)
    o = agg_ref[0] * rl[:, None] + agg_ref[1] * ru[:, None] + sk_ref[...]
    out_ref[...] = jnp.maximum(o, 0.0)


_post = pl.pallas_call(
    _post_body,
    grid=((N + BN - 1) // BN,),
    in_specs=[
        pl.BlockSpec((2, BN, D), lambda i: (0, i, 0)),
        pl.BlockSpec((2, NS, BN), lambda i: (0, 0, i)),
        pl.BlockSpec((BN, D), lambda i: (i, 0)),
    ],
    out_specs=pl.BlockSpec((BN, D), lambda i: (i, 0)),
    out_shape=jax.ShapeDtypeStruct((N, D), jnp.float32),
)


@jax.jit
def kernel(x, lower_edge_index, upper_edge_index,
           W_low, a_src_low, a_dst_low,
           W_up, a_src_up, a_dst_up,
           W_skip):
    av = jnp.concatenate([
        a_src_low.reshape(1, D)[:, _PERM], a_dst_low.reshape(1, D)[:, _PERM],
        a_src_up.reshape(1, D)[:, _PERM], a_dst_up.reshape(1, D)[:, _PERM]],
        axis=0)
    xw, sk, st = _pre(x, W_low[:, _PERM], W_up[:, _PERM], W_skip, av)
    elow5 = lower_edge_index.reshape(2, NS, NSB1, SB1E)
    eup5 = upper_edge_index.reshape(2, NS, NSB1, SB1E)
    elow6 = lower_edge_index.reshape(2, NS, NSB, SBC, K)
    eup6 = upper_edge_index.reshape(2, NS, NSB, SBC, K)
    w, den = _sc1(elow5, eup5, st)
    agg = _sc2(elow6, eup6, w, xw)
    return _post(agg, den, sk)
